# Initial kernel scaffold; baseline (speedup 1.0000x reference)
import functools

import jax
import jax.numpy as jnp
from jax import lax
from jax.experimental import pallas as pl
from jax.experimental.pallas import tpu as pltpu
from jax.experimental.pallas import tpu_sc as plsc

_B = 1024
_NC, _NS, _LANES = 2, 16, 16
_NW = _NC * _NS
_BPW = _B // _NW
_NBR_PER_W = _BPW * 54

_INTERPRET = False

_SPATIAL_OFFS = [(dx, dy, dz) for dx in (-1, 0, 1) for dy in (-1, 0, 1)
                 for dz in (-1, 0, 1)]


def _sc_body(px_h, py_h, pz_h, t_h, t0_h, t1_h, ts1_h, ts2_h,
             f0_h, f1_h, tfm_h, tf0_h, tfp_h, nf_h, lin2_h, lin3_h,
             px_v, py_v, pz_v, t_v,
             lin0_v, lin1_v, lin2_v, lin3_v, tsm_v, ts0_v, tsp_v,
             nbr_v, f0_v, f1_v, tfm_v, tf0_v, tfp_v, nf_v, sem):
    wid = lax.axis_index("s") * _NC + lax.axis_index("c")
    base = wid * _BPW

    pltpu.sync_copy(px_h.at[pl.ds(base, _BPW)], px_v)
    pltpu.sync_copy(py_h.at[pl.ds(base, _BPW)], py_v)
    pltpu.sync_copy(pz_h.at[pl.ds(base, _BPW)], pz_v)
    pltpu.sync_copy(t_h.at[pl.ds(base, _BPW)], t_v)

    iot = lax.iota(jnp.int32, _LANES)
    for c in range(_BPW // _LANES):
        sl = pl.ds(c * _LANES, _LANES)
        px = px_v[sl]
        py = py_v[sl]
        pz = pz_v[sl]
        tt = t_v[sl]
        x0 = (px * 127.0).astype(jnp.int32)
        y0 = (py * 127.0).astype(jnp.int32)
        z0 = (pz * 127.0).astype(jnp.int32)
        x1 = (px * 63.0).astype(jnp.int32)
        y1 = (py * 63.0).astype(jnp.int32)
        z1 = (pz * 63.0).astype(jnp.int32)
        x2 = (px * 31.0).astype(jnp.int32)
        y2 = (py * 31.0).astype(jnp.int32)
        z2 = (pz * 31.0).astype(jnp.int32)
        x3 = (px * 15.0).astype(jnp.int32)
        y3 = (py * 15.0).astype(jnp.int32)
        z3 = (pz * 15.0).astype(jnp.int32)
        t127 = (tt * 127.0).astype(jnp.int32)
        t64 = (tt * 63.0).astype(jnp.int32)

        lin0_v[sl] = (x0 * 128 + y0) * 128 + z0
        lin1_v[sl] = (x1 * 64 + y1) * 64 + z1
        lin2_v[sl] = (x2 * 32 + y2) * 32 + z2
        lin3 = (x3 * 16 + y3) * 16 + z3
        lin3_v[sl] = lin3

        base3 = lin3 * 64
        tsm_v[sl] = base3 + ((t64 + 63) & 63)
        ts0_v[sl] = base3 + t64
        tsp_v[sl] = base3 + ((t64 + 65) & 63)

        xs = {-1: (x3 + 127) & 127, 0: x3, 1: (x3 + 129) & 127}
        ys = {-1: (y3 + 127) & 127, 0: y3, 1: (y3 + 129) & 127}
        zs = {-1: (z3 + 127) & 127, 0: z3, 1: (z3 + 129) & 127}
        tms = {-1: (t127 + 127) & 127, 1: (t127 + 129) & 127}
        pbase = iot * 54 + c * _LANES * 54
        for s, (dx, dy, dz) in enumerate(_SPATIAL_OFFS):
            sp = ((xs[dx] * 128 + ys[dy]) * 128 + zs[dz]) * 128
            for dti, dt in enumerate((-1, 1)):
                lin = sp + tms[dt]
                p = pbase + (s * 2 + dti)
                plsc.store_scatter(nbr_v, [p >> 6, p & 63], lin)

    cps = [
        pltpu.async_copy(t0_h.at[lin0_v], f0_v, sem),
        pltpu.async_copy(t1_h.at[lin1_v], f1_v, sem),
        pltpu.async_copy(ts1_h.at[tsm_v], tfm_v, sem),
        pltpu.async_copy(ts1_h.at[ts0_v], tf0_v, sem),
        pltpu.async_copy(ts1_h.at[tsp_v], tfp_v, sem),
    ]
    for r in range(27):
        cps.append(pltpu.async_copy(
            ts2_h.at[nbr_v.at[r]], nf_v.at[pl.ds(r * 64, 64)], sem))
    for cp in cps:
        cp.wait()

    pltpu.sync_copy(f0_v, f0_h.at[pl.ds(base, _BPW)])
    pltpu.sync_copy(f1_v, f1_h.at[pl.ds(base, _BPW)])
    pltpu.sync_copy(tfm_v, tfm_h.at[pl.ds(base, _BPW)])
    pltpu.sync_copy(tf0_v, tf0_h.at[pl.ds(base, _BPW)])
    pltpu.sync_copy(tfp_v, tfp_h.at[pl.ds(base, _BPW)])
    pltpu.sync_copy(nf_v, nf_h.at[pl.ds(base * 54, _NBR_PER_W)])
    pltpu.sync_copy(lin2_v, lin2_h.at[pl.ds(base, _BPW)])
    pltpu.sync_copy(lin3_v, lin3_h.at[pl.ds(base, _BPW)])


def _sc_gather(px, py, pz, t, t0r, t1r, ts1r, ts2r):
    f32, i32 = jnp.float32, jnp.int32
    out_type = (
        jax.ShapeDtypeStruct((_B, 64), f32),
        jax.ShapeDtypeStruct((_B, 512), f32),
        jax.ShapeDtypeStruct((_B, 64), f32),
        jax.ShapeDtypeStruct((_B, 64), f32),
        jax.ShapeDtypeStruct((_B, 64), f32),
        jax.ShapeDtypeStruct((_B * 54, 4), f32),
        jax.ShapeDtypeStruct((_B,), i32),
        jax.ShapeDtypeStruct((_B,), i32),
    )
    scratch = [
        pltpu.VMEM((_BPW,), f32), pltpu.VMEM((_BPW,), f32),
        pltpu.VMEM((_BPW,), f32), pltpu.VMEM((_BPW,), f32),
        pltpu.VMEM((_BPW,), i32), pltpu.VMEM((_BPW,), i32),
        pltpu.VMEM((_BPW,), i32), pltpu.VMEM((_BPW,), i32),
        pltpu.VMEM((_BPW,), i32), pltpu.VMEM((_BPW,), i32),
        pltpu.VMEM((_BPW,), i32),
        pltpu.VMEM((27, 64), i32),
        pltpu.VMEM((_BPW, 64), f32), pltpu.VMEM((_BPW, 512), f32),
        pltpu.VMEM((_BPW, 64), f32), pltpu.VMEM((_BPW, 64), f32),
        pltpu.VMEM((_BPW, 64), f32),
        pltpu.VMEM((_NBR_PER_W, 4), f32),
        pltpu.SemaphoreType.DMA,
    ]
    mesh = plsc.VectorSubcoreMesh(core_axis_name="c", subcore_axis_name="s",
                                  num_cores=_NC, num_subcores=_NS)
    fn = pl.kernel(_sc_body, out_type, mesh=mesh, scratch_types=scratch,
                   interpret=_INTERPRET)
    return fn(px, py, pz, t, t0r, t1r, ts1r, ts2r)


def _gmm_body(nk, nb, rb, kc, lin_ref, tbl_ref, w_ref, out_ref, buf, sem):
    k = pl.program_id(0)
    b = pl.program_id(1)
    s = k * nb + b

    def issue(kk, bb, ib):
        for i in range(rb):
            row = lin_ref[bb * rb + i]
            pltpu.make_async_copy(
                tbl_ref.at[pl.ds(row, 1), pl.ds(kk * kc, kc)],
                buf.at[ib, pl.ds(i, 1), :], sem).start()

    def drain(kk, bb, ib):
        for i in range(rb):
            row = lin_ref[bb * rb + i]
            pltpu.make_async_copy(
                tbl_ref.at[pl.ds(row, 1), pl.ds(kk * kc, kc)],
                buf.at[ib, pl.ds(i, 1), :], sem).wait()

    @pl.when(s == 0)
    def _():
        issue(k, b, 0)

    @pl.when(s + 1 < nk * nb)
    def _():
        sn = s + 1
        issue(sn // nb, sn % nb, sn % 2)

    drain(k, b, s % 2)
    w = w_ref[...]

    @pl.when(s % 2 == 0)
    def _():
        out_ref[0] = jnp.dot(jnp.maximum(buf[0], 0.0), w,
                             preferred_element_type=jnp.float32)

    @pl.when(s % 2 == 1)
    def _():
        out_ref[0] = jnp.dot(jnp.maximum(buf[1], 0.0), w,
                             preferred_element_type=jnp.float32)


def _gather_matmul(lin, tbl2d, wseg, nk, kc, rb):
    nb = _B // rb
    grid_spec = pltpu.PrefetchScalarGridSpec(
        num_scalar_prefetch=1,
        grid=(nk, nb),
        in_specs=[
            pl.BlockSpec(memory_space=pltpu.ANY),
            pl.BlockSpec((kc, 128), lambda k, b, lin_: (k, 0)),
        ],
        out_specs=pl.BlockSpec((1, rb, 128), lambda k, b, lin_: (k, b, 0)),
        scratch_shapes=[
            pltpu.VMEM((2, rb, kc), jnp.float32),
            pltpu.SemaphoreType.DMA,
        ],
    )
    fn = pl.pallas_call(
        functools.partial(_gmm_body, nk, nb, rb, kc),
        grid_spec=grid_spec,
        out_shape=jax.ShapeDtypeStruct((nk, _B, 128), jnp.float32),
        interpret=_INTERPRET,
    )
    return fn(lin, tbl2d, wseg)


def _tail_body(h3p_ref, h2p_ref, f0_ref, f1_ref, tfm_ref, tf0_ref, tfp_ref,
               nf_ref, w0_ref, w1_ref, wtm_ref, wt0_ref, wtp_ref, wnf_ref,
               dir_ref, t_ref, wmid_ref, lng_ref, lnb_ref, wfin_ref,
               bfin_ref, out_ref):
    f32 = jnp.float32

    def rmm(x_ref, w_ref):
        return jnp.dot(jnp.maximum(x_ref[...], 0.0), w_ref[...],
                       preferred_element_type=f32)

    h = (h3p_ref[0] + h3p_ref[1] + h3p_ref[2] + h3p_ref[3] + h2p_ref[0]
         + rmm(f0_ref, w0_ref) + rmm(f1_ref, w1_ref) + rmm(tfm_ref, wtm_ref)
         + rmm(tf0_ref, wt0_ref) + rmm(tfp_ref, wtp_ref)
         + rmm(nf_ref, wnf_ref))
    wmid = wmid_ref[...]
    ff = (jnp.dot(h, wmid[0:128, :], preferred_element_type=f32)
          + jnp.dot(dir_ref[...], wmid[128:131, :], preferred_element_type=f32)
          + t_ref[...] * wmid[131, :][None, :])
    mu = jnp.mean(ff, axis=-1, keepdims=True)
    d = ff - mu
    var = jnp.mean(d * d, axis=-1, keepdims=True)
    ffn = d * lax.rsqrt(var + 1e-5) * lng_ref[...] + lnb_ref[...]
    out_ref[...] = (jnp.dot(ffn, wfin_ref[...], preferred_element_type=f32)
                    + bfin_ref[...])


def _tail(h3p, h2p, f0, f1, tfm, tf0, tfp, nf, w0, w1, wtm, wt0, wtp, wnf,
          dirs, t, wmid, lng, lnb, wfin, bfin):
    fn = pl.pallas_call(
        _tail_body,
        out_shape=jax.ShapeDtypeStruct((_B, 4), jnp.float32),
        interpret=_INTERPRET,
    )
    return fn(h3p, h2p, f0, f1, tfm, tf0, tfp, nf, w0, w1, wtm, wt0, wtp,
              wnf, dirs, t, wmid, lng, lnb, wfin, bfin)


def kernel(pos, dir, t, table0, table1, table2, table3, ts_table1, ts_table2,
           W_down, W_mid, ln_g, ln_b, W_fin, b_fin):
    px = pos[:, 0]
    py = pos[:, 1]
    pz = pos[:, 2]
    t0r = table0.reshape(128 * 128 * 128, 64)
    t1r = table1.reshape(64 * 64 * 64, 512)
    t2r = table2.reshape(32 * 32 * 32, 4096)
    t3r = table3.reshape(16 * 16 * 16, 32768)
    ts1r = ts_table1.reshape(16 * 16 * 16 * 64, 64)
    ts2r = ts_table2.reshape(128 * 128 * 128 * 128, 4)

    f0, f1, tfm, tf0, tfp, nf, lin2, lin3 = _sc_gather(
        px, py, pz, t, t0r, t1r, ts1r, ts2r)

    h2p = _gather_matmul(lin2, t2r, W_down[576:4672], nk=1, kc=4096, rb=64)
    h3p = _gather_matmul(lin3, t3r, W_down[4672:37440], nk=4, kc=8192, rb=64)

    return _tail(
        h3p, h2p, f0, f1, tfm, tf0, tfp, nf.reshape(_B, 216),
        W_down[0:64], W_down[64:576],
        W_down[37440:37504], W_down[37504:37568], W_down[37568:37632],
        W_down[37632:37848],
        dir, t.reshape(_B, 1), W_mid, ln_g.reshape(1, 132),
        ln_b.reshape(1, 132), W_fin, b_fin.reshape(1, 4))


# speedup vs baseline: 12.8183x; 12.8183x over previous
import functools

import jax
import jax.numpy as jnp
from jax import lax
from jax.experimental import pallas as pl
from jax.experimental.pallas import tpu as pltpu
from jax.experimental.pallas import tpu_sc as plsc

_B = 1024
_NC, _NS, _LANES = 2, 16, 16
_NW = _NC * _NS
_BPW = _B // _NW

_INTERPRET = False

_SPATIAL_OFFS = [(dx, dy, dz) for dx in (-1, 0, 1) for dy in (-1, 0, 1)
                 for dz in (-1, 0, 1)]


def _sc_body(px_h, py_h, pz_h, t_h, t0f_h, ts2t_h,
             f0g_h, nfT_h, lin1_h, lin2_h, lin3_h, srow_h,
             px_v, py_v, pz_v, t_v,
             lin1_v, lin2_v, lin3_v, srow_v, tm_v, tp_v,
             widx_v, nbrg_v, f0g_v, nfw_v, nfT_v, sem, sem2):
    wid = lax.axis_index("s") * _NC + lax.axis_index("c")
    base = wid * _BPW

    pltpu.sync_copy(px_h.at[pl.ds(base, _BPW)], px_v)
    pltpu.sync_copy(py_h.at[pl.ds(base, _BPW)], py_v)
    pltpu.sync_copy(pz_h.at[pl.ds(base, _BPW)], pz_v)
    pltpu.sync_copy(t_h.at[pl.ds(base, _BPW)], t_v)

    for c in range(_BPW // _LANES):
        sl = pl.ds(c * _LANES, _LANES)
        px = px_v[sl]
        py = py_v[sl]
        pz = pz_v[sl]
        tt = t_v[sl]
        x0 = (px * 127.0).astype(jnp.int32)
        y0 = (py * 127.0).astype(jnp.int32)
        z0 = (pz * 127.0).astype(jnp.int32)
        x1 = (px * 63.0).astype(jnp.int32)
        y1 = (py * 63.0).astype(jnp.int32)
        z1 = (pz * 63.0).astype(jnp.int32)
        x2 = (px * 31.0).astype(jnp.int32)
        y2 = (py * 31.0).astype(jnp.int32)
        z2 = (pz * 31.0).astype(jnp.int32)
        x3 = (px * 15.0).astype(jnp.int32)
        y3 = (py * 15.0).astype(jnp.int32)
        z3 = (pz * 15.0).astype(jnp.int32)
        t127 = (tt * 127.0).astype(jnp.int32)
        t64 = (tt * 63.0).astype(jnp.int32)

        lin1_v[sl] = (x1 * 64 + y1) * 64 + z1
        lin2_v[sl] = (x2 * 32 + y2) * 32 + z2
        lin3 = (x3 * 16 + y3) * 16 + z3
        lin3_v[sl] = lin3
        srow_v[sl] = jnp.minimum(jnp.maximum(t64 - 1, 0), 61)
        tm_v[sl] = (t127 + 127) & 127
        tp_v[sl] = (t127 + 129) & 127

        base0 = (x0 * 128 + y0) * 8192 + z0
        for f in range(64):
            p = f * _BPW + c * _LANES
            widx_v[p >> 7, pl.ds(p & 127, _LANES)] = base0 + f * 128

        xs = {-1: (x3 + 127) & 127, 0: x3, 1: (x3 + 129) & 127}
        ys = {-1: (y3 + 127) & 127, 0: y3, 1: (y3 + 129) & 127}
        zs = {-1: (z3 + 127) & 127, 0: z3, 1: (z3 + 129) & 127}
        for s, (dx, dy, dz) in enumerate(_SPATIAL_OFFS):
            spb = ((xs[dx] * 128 + ys[dy]) * 128 + zs[dz]) * 4
            for f in range(4):
                nbrg_v[s, pl.ds(f * _BPW + c * _LANES, _LANES)] = spb + f

    iot = lax.iota(jnp.int32, _LANES)
    f0cps = [pltpu.async_copy(t0f_h.at[widx_v.at[r]],
                              f0g_v.at[pl.ds(r * 128, 128)], sem2)
             for r in range(16)]

    def nf_fire(s, ib):
        return pltpu.async_copy(ts2t_h.at[nbrg_v.at[s]], nfw_v.at[ib], sem)

    def nf_extract(s, ib):
        for c in range(_BPW // _LANES):
            tm = tm_v[pl.ds(c * _LANES, _LANES)]
            tp = tp_v[pl.ds(c * _LANES, _LANES)]
            for f in range(4):
                rows = f * _BPW + c * _LANES + iot
                vm = plsc.load_gather(nfw_v.at[ib], [rows, tm])
                vp = plsc.load_gather(nfw_v.at[ib], [rows, tp])
                nfT_v[(s * 2 + 0) * 4 + f, pl.ds(c * _LANES, _LANES)] = vm
                nfT_v[(s * 2 + 1) * 4 + f, pl.ds(c * _LANES, _LANES)] = vp

    cps = {0: nf_fire(0, 0)}
    for s in range(27):
        if s + 1 < 27:
            cps[s + 1] = nf_fire(s + 1, (s + 1) % 2)
        cps[s].wait()
        nf_extract(s, s % 2)

    for cp in f0cps:
        cp.wait()

    pltpu.sync_copy(f0g_v, f0g_h.at[wid])
    pltpu.sync_copy(nfT_v, nfT_h.at[:, pl.ds(base, _BPW)])
    pltpu.sync_copy(lin1_v, lin1_h.at[pl.ds(base, _BPW)])
    pltpu.sync_copy(lin2_v, lin2_h.at[pl.ds(base, _BPW)])
    pltpu.sync_copy(lin3_v, lin3_h.at[pl.ds(base, _BPW)])
    pltpu.sync_copy(srow_v, srow_h.at[pl.ds(base, _BPW)])


def _sc_gather(px, py, pz, t, t0f, ts2t):
    f32, i32 = jnp.float32, jnp.int32
    out_type = (
        jax.ShapeDtypeStruct((_NW, 64 * _BPW), f32),
        jax.ShapeDtypeStruct((216, _B), f32),
        jax.ShapeDtypeStruct((_B,), i32),
        jax.ShapeDtypeStruct((_B,), i32),
        jax.ShapeDtypeStruct((_B,), i32),
        jax.ShapeDtypeStruct((_B,), i32),
    )
    scratch = [
        pltpu.VMEM((_BPW,), f32), pltpu.VMEM((_BPW,), f32),
        pltpu.VMEM((_BPW,), f32), pltpu.VMEM((_BPW,), f32),
        pltpu.VMEM((_BPW,), i32), pltpu.VMEM((_BPW,), i32),
        pltpu.VMEM((_BPW,), i32), pltpu.VMEM((_BPW,), i32),
        pltpu.VMEM((_BPW,), i32), pltpu.VMEM((_BPW,), i32),
        pltpu.VMEM((16, 128), i32),
        pltpu.VMEM((27, 128), i32),
        pltpu.VMEM((64 * _BPW,), f32),
        pltpu.VMEM((2, 128, 128), f32),
        pltpu.VMEM((216, _BPW), f32),
        pltpu.SemaphoreType.DMA, pltpu.SemaphoreType.DMA,
    ]
    mesh = plsc.VectorSubcoreMesh(core_axis_name="c", subcore_axis_name="s",
                                  num_cores=_NC, num_subcores=_NS)
    fn = pl.kernel(_sc_body, out_type, mesh=mesh, scratch_types=scratch,
                   compiler_params=pltpu.CompilerParams(
                       use_tc_tiling_on_sc=False, needs_layout_passes=False),
                   interpret=_INTERPRET)
    return fn(px, py, pz, t, t0f, ts2t)


def _a2_body(lin2_ref, lin1_ref, lin3_ref, srow_ref,
             t2_ref, t1_ref, ts1_ref, w2_ref, w1_ref,
             h2_ref, tsraw_ref, f2b, f1b, tsb, sem):
    rb = f2b.shape[1]
    b = pl.program_id(0)
    nb = pl.num_programs(0)

    def cps(bb, ib):
        out = []
        for i in range(rb):
            r2 = lin2_ref[bb * rb + i]
            r1 = lin1_ref[bb * rb + i]
            c3 = lin3_ref[bb * rb + i]
            sr = srow_ref[bb * rb + i]
            out.append(pltpu.make_async_copy(
                t2_ref.at[pl.ds(r2, 1), :], f2b.at[ib, pl.ds(i, 1), :], sem))
            out.append(pltpu.make_async_copy(
                t1_ref.at[pl.ds(r1, 1), :], f1b.at[ib, pl.ds(i, 1), :], sem))
            out.append(pltpu.make_async_copy(
                ts1_ref.at[pl.ds(c3, 1), pl.ds(sr, 3), :],
                tsb.at[ib, pl.ds(i, 1), pl.ds(0, 3), :], sem))
            out.append(pltpu.make_async_copy(
                ts1_ref.at[pl.ds(c3, 1), pl.ds(63, 1), :],
                tsb.at[ib, pl.ds(i, 1), pl.ds(3, 1), :], sem))
        return out

    @pl.when(b == 0)
    def _():
        for cp in cps(b, 0):
            cp.start()

    @pl.when(b + 1 < nb)
    def _():
        for cp in cps(b + 1, (b + 1) % 2):
            cp.start()

    for cp in cps(b, b % 2):
        cp.wait()

    def compute(ib):
        h2_ref[...] = (
            jnp.dot(jnp.maximum(f2b[ib], 0.0), w2_ref[...],
                    preferred_element_type=jnp.float32)
            + jnp.dot(jnp.maximum(f1b[ib], 0.0), w1_ref[...],
                      preferred_element_type=jnp.float32))
        tsraw_ref[...] = tsb[ib]

    @pl.when(b % 2 == 0)
    def _():
        compute(0)

    @pl.when(b % 2 == 1)
    def _():
        compute(1)


def _a2_call(lin2, lin1, lin3, srow, t2r, t1r, ts1c, w2, w1, rb=64):
    nb = _B // rb
    grid_spec = pltpu.PrefetchScalarGridSpec(
        num_scalar_prefetch=4,
        grid=(nb,),
        in_specs=[
            pl.BlockSpec(memory_space=pltpu.MemorySpace.HBM),
            pl.BlockSpec(memory_space=pltpu.MemorySpace.HBM),
            pl.BlockSpec(memory_space=pltpu.MemorySpace.HBM),
            pl.BlockSpec((4096, 128), lambda b, *_: (0, 0)),
            pl.BlockSpec((512, 128), lambda b, *_: (0, 0)),
        ],
        out_specs=[
            pl.BlockSpec((rb, 128), lambda b, *_: (b, 0)),
            pl.BlockSpec((rb, 4, 64), lambda b, *_: (b, 0, 0)),
        ],
        scratch_shapes=[
            pltpu.VMEM((2, rb, 4096), jnp.float32),
            pltpu.VMEM((2, rb, 512), jnp.float32),
            pltpu.VMEM((2, rb, 4, 64), jnp.float32),
            pltpu.SemaphoreType.DMA,
        ],
    )
    fn = pl.pallas_call(
        _a2_body,
        grid_spec=grid_spec,
        out_shape=[
            jax.ShapeDtypeStruct((_B, 128), jnp.float32),
            jax.ShapeDtypeStruct((_B, 4, 64), jnp.float32),
        ],
        interpret=_INTERPRET,
    )
    return fn(lin2, lin1, lin3, srow, t2r, t1r, ts1c, w2, w1)


def _gmm_body(nk, nb, rb, kc, lin_ref, tbl_ref, w_ref, out_ref, buf, sem):
    k = pl.program_id(0)
    b = pl.program_id(1)
    s = k * nb + b

    def cps(kk, bb, ib):
        out = []
        for i in range(rb):
            row = lin_ref[bb * rb + i]
            out.append(pltpu.make_async_copy(
                tbl_ref.at[pl.ds(row, 1), pl.ds(kk * kc, kc)],
                buf.at[ib, pl.ds(i, 1), :], sem))
        return out

    @pl.when(s == 0)
    def _():
        for cp in cps(k, b, 0):
            cp.start()

    @pl.when(s + 1 < nk * nb)
    def _():
        sn = s + 1
        for cp in cps(sn // nb, sn % nb, sn % 2):
            cp.start()

    for cp in cps(k, b, s % 2):
        cp.wait()
    w = w_ref[...]

    @pl.when(s % 2 == 0)
    def _():
        out_ref[0] = jnp.dot(jnp.maximum(buf[0], 0.0), w,
                             preferred_element_type=jnp.float32)

    @pl.when(s % 2 == 1)
    def _():
        out_ref[0] = jnp.dot(jnp.maximum(buf[1], 0.0), w,
                             preferred_element_type=jnp.float32)


def _gather_matmul(lin, tbl2d, wseg, nk, kc, rb):
    nb = _B // rb
    grid_spec = pltpu.PrefetchScalarGridSpec(
        num_scalar_prefetch=1,
        grid=(nk, nb),
        in_specs=[
            pl.BlockSpec(memory_space=pltpu.MemorySpace.HBM),
            pl.BlockSpec((kc, 128), lambda k, b, lin_: (k, 0)),
        ],
        out_specs=pl.BlockSpec((1, rb, 128), lambda k, b, lin_: (k, b, 0)),
        scratch_shapes=[
            pltpu.VMEM((2, rb, kc), jnp.float32),
            pltpu.SemaphoreType.DMA,
        ],
    )
    fn = pl.pallas_call(
        functools.partial(_gmm_body, nk, nb, rb, kc),
        grid_spec=grid_spec,
        out_shape=jax.ShapeDtypeStruct((nk, _B, 128), jnp.float32),
        interpret=_INTERPRET,
    )
    return fn(lin, tbl2d, wseg)


def _tail_body(h3p_ref, h2_ref, f0T_ref, nfT_ref, tsraw_ref,
               w0_ref, wtm_ref, wt0_ref, wtp_ref, wnf_ref,
               dir_ref, t_ref, wmid_ref, lng_ref, lnb_ref, wfin_ref,
               bfin_ref, out_ref):
    f32 = jnp.float32

    def rmm(x, w_ref):
        return jnp.dot(jnp.maximum(x, 0.0), w_ref[...],
                       preferred_element_type=f32)

    def rtmm(xT_ref, w_ref):
        return lax.dot_general(jnp.maximum(xT_ref[...], 0.0), w_ref[...],
                               (((0,), (0,)), ((), ())),
                               preferred_element_type=f32)

    t64 = (t_ref[...] * 63.0).astype(jnp.int32)
    wrap = t64 == 0
    raw0 = tsraw_ref[:, 0, :]
    raw1 = tsraw_ref[:, 1, :]
    raw2 = tsraw_ref[:, 2, :]
    raw3 = tsraw_ref[:, 3, :]
    tfm = jnp.where(wrap, raw3, raw0)
    tf0 = jnp.where(wrap, raw0, raw1)
    tfp = jnp.where(wrap, raw1, raw2)

    h = (h3p_ref[0] + h3p_ref[1] + h3p_ref[2] + h3p_ref[3] + h2_ref[...]
         + rtmm(f0T_ref, w0_ref) + rtmm(nfT_ref, wnf_ref)
         + rmm(tfm, wtm_ref) + rmm(tf0, wt0_ref) + rmm(tfp, wtp_ref))
    wmid = wmid_ref[...]
    ff = (jnp.dot(h, wmid[0:128, :], preferred_element_type=f32)
          + jnp.dot(dir_ref[...], wmid[128:131, :], preferred_element_type=f32)
          + t_ref[...] * wmid[131, :][None, :])
    mu = jnp.mean(ff, axis=-1, keepdims=True)
    d = ff - mu
    var = jnp.mean(d * d, axis=-1, keepdims=True)
    ffn = d * lax.rsqrt(var + 1e-5) * lng_ref[...] + lnb_ref[...]
    out_ref[...] = (jnp.dot(ffn, wfin_ref[...], preferred_element_type=f32)
                    + bfin_ref[...])


def _tail(h3p, h2, f0T, nfT, tsraw, w0, wtm, wt0, wtp, wnf,
          dirs, t, wmid, lng, lnb, wfin, bfin):
    fn = pl.pallas_call(
        _tail_body,
        out_shape=jax.ShapeDtypeStruct((_B, 4), jnp.float32),
        interpret=_INTERPRET,
    )
    return fn(h3p, h2, f0T, nfT, tsraw, w0, wtm, wt0, wtp, wnf,
              dirs, t, wmid, lng, lnb, wfin, bfin)


def kernel(pos, dir, t, table0, table1, table2, table3, ts_table1, ts_table2,
           W_down, W_mid, ln_g, ln_b, W_fin, b_fin):
    px = pos[:, 0]
    py = pos[:, 1]
    pz = pos[:, 2]
    t0f = table0.transpose(0, 1, 3, 2).reshape(-1)
    ts2t = ts_table2.transpose(0, 1, 2, 4, 3).reshape(128 * 128 * 128 * 4, 128)
    t1r = table1.reshape(64 * 64 * 64, 512)
    t2r = table2.reshape(32 * 32 * 32, 4096)
    t3r = table3.reshape(16 * 16 * 16, 32768)
    ts1c = ts_table1.reshape(16 * 16 * 16, 64, 64)

    f0g, nfT, lin1, lin2, lin3, srow = _sc_gather(px, py, pz, t, t0f, ts2t)
    f0T = f0g.reshape(_NW, 64, _BPW).transpose(1, 0, 2).reshape(64, _B)

    h2, tsraw = _a2_call(lin2, lin1, lin3, srow, t2r, t1r, ts1c,
                         W_down[576:4672], W_down[64:576])
    h3p = _gather_matmul(lin3, t3r, W_down[4672:37440], nk=4, kc=8192, rb=64)

    return _tail(
        h3p, h2, f0T, nfT, tsraw,
        W_down[0:64],
        W_down[37440:37504], W_down[37504:37568], W_down[37568:37632],
        W_down[37632:37848],
        dir, t.reshape(_B, 1), W_mid, ln_g.reshape(1, 132),
        ln_b.reshape(1, 132), W_fin, b_fin.reshape(1, 4))


# speedup vs baseline: 15.9487x; 1.2442x over previous
import functools

import jax
import jax.numpy as jnp
from jax import lax
from jax.experimental import pallas as pl
from jax.experimental.pallas import tpu as pltpu
from jax.experimental.pallas import tpu_sc as plsc

_B = 1024
_NC, _NS, _LANES = 2, 16, 16
_NW = _NC * _NS
_BPW = _B // _NW

_INTERPRET = False

_SPATIAL_OFFS = [(dx, dy, dz) for dx in (-1, 0, 1) for dy in (-1, 0, 1)
                 for dz in (-1, 0, 1)]


def _sc_body(px_h, py_h, pz_h, t_h, t0f_h, ts2t_h,
             f0g_h, nfT_h, lin1_h, lin2_h, lin3_h, srow_h,
             px_v, py_v, pz_v, t_v,
             lin1_v, lin2_v, lin3_v, srow_v, tm_v, tp_v,
             widx_v, nbrg_v, f0g_v, nfw_v, nfT_v, sem, sem2):
    wid = lax.axis_index("s") * _NC + lax.axis_index("c")
    base = wid * _BPW

    pltpu.sync_copy(px_h.at[pl.ds(base, _BPW)], px_v)
    pltpu.sync_copy(py_h.at[pl.ds(base, _BPW)], py_v)
    pltpu.sync_copy(pz_h.at[pl.ds(base, _BPW)], pz_v)
    pltpu.sync_copy(t_h.at[pl.ds(base, _BPW)], t_v)

    for c in range(_BPW // _LANES):
        sl = pl.ds(c * _LANES, _LANES)
        px = px_v[sl]
        py = py_v[sl]
        pz = pz_v[sl]
        tt = t_v[sl]
        x0 = (px * 127.0).astype(jnp.int32)
        y0 = (py * 127.0).astype(jnp.int32)
        z0 = (pz * 127.0).astype(jnp.int32)
        x1 = (px * 63.0).astype(jnp.int32)
        y1 = (py * 63.0).astype(jnp.int32)
        z1 = (pz * 63.0).astype(jnp.int32)
        x2 = (px * 31.0).astype(jnp.int32)
        y2 = (py * 31.0).astype(jnp.int32)
        z2 = (pz * 31.0).astype(jnp.int32)
        x3 = (px * 15.0).astype(jnp.int32)
        y3 = (py * 15.0).astype(jnp.int32)
        z3 = (pz * 15.0).astype(jnp.int32)
        t127 = (tt * 127.0).astype(jnp.int32)
        t64 = (tt * 63.0).astype(jnp.int32)

        lin1_v[sl] = (x1 * 64 + y1) * 64 + z1
        lin2_v[sl] = (x2 * 32 + y2) * 32 + z2
        lin3 = (x3 * 16 + y3) * 16 + z3
        lin3_v[sl] = lin3
        srow_v[sl] = jnp.minimum(jnp.maximum(t64 - 1, 0), 61)
        tm_v[sl] = (t127 + 127) & 127
        tp_v[sl] = (t127 + 129) & 127

        base0 = (x0 * 128 + y0) * 8192 + z0
        for f in range(64):
            p = f * _BPW + c * _LANES
            widx_v[p >> 7, pl.ds(p & 127, _LANES)] = base0 + f * 128

        xs = {-1: (x3 + 127) & 127, 0: x3, 1: (x3 + 129) & 127}
        ys = {-1: (y3 + 127) & 127, 0: y3, 1: (y3 + 129) & 127}
        zs = {-1: (z3 + 127) & 127, 0: z3, 1: (z3 + 129) & 127}
        for s, (dx, dy, dz) in enumerate(_SPATIAL_OFFS):
            spb = ((xs[dx] * 128 + ys[dy]) * 128 + zs[dz]) * 4
            for f in range(4):
                nbrg_v[s, pl.ds(f * _BPW + c * _LANES, _LANES)] = spb + f

    iot = lax.iota(jnp.int32, _LANES)
    f0cps = [pltpu.async_copy(t0f_h.at[widx_v.at[r]],
                              f0g_v.at[pl.ds(r * 128, 128)], sem2)
             for r in range(16)]

    def nf_fire(s, ib):
        return pltpu.async_copy(ts2t_h.at[nbrg_v.at[s]], nfw_v.at[ib], sem)

    def nf_extract(s, ib):
        for c in range(_BPW // _LANES):
            tm = tm_v[pl.ds(c * _LANES, _LANES)]
            tp = tp_v[pl.ds(c * _LANES, _LANES)]
            for f in range(4):
                rows = f * _BPW + c * _LANES + iot
                vm = plsc.load_gather(nfw_v.at[ib], [rows, tm])
                vp = plsc.load_gather(nfw_v.at[ib], [rows, tp])
                nfT_v[(s * 2 + 0) * 4 + f, pl.ds(c * _LANES, _LANES)] = vm
                nfT_v[(s * 2 + 1) * 4 + f, pl.ds(c * _LANES, _LANES)] = vp

    cps = {0: nf_fire(0, 0)}
    for s in range(27):
        if s + 1 < 27:
            cps[s + 1] = nf_fire(s + 1, (s + 1) % 2)
        cps[s].wait()
        nf_extract(s, s % 2)

    for cp in f0cps:
        cp.wait()

    pltpu.sync_copy(f0g_v, f0g_h.at[wid])
    pltpu.sync_copy(nfT_v, nfT_h.at[:, pl.ds(base, _BPW)])
    pltpu.sync_copy(lin1_v, lin1_h.at[pl.ds(base, _BPW)])
    pltpu.sync_copy(lin2_v, lin2_h.at[pl.ds(base, _BPW)])
    pltpu.sync_copy(lin3_v, lin3_h.at[pl.ds(base, _BPW)])
    pltpu.sync_copy(srow_v, srow_h.at[pl.ds(base, _BPW)])


def _sc_gather(px, py, pz, t, t0f, ts2t):
    f32, i32 = jnp.float32, jnp.int32
    out_type = (
        jax.ShapeDtypeStruct((_NW, 64 * _BPW), f32),
        jax.ShapeDtypeStruct((216, _B), f32),
        jax.ShapeDtypeStruct((_B,), i32),
        jax.ShapeDtypeStruct((_B,), i32),
        jax.ShapeDtypeStruct((_B,), i32),
        jax.ShapeDtypeStruct((_B,), i32),
    )
    scratch = [
        pltpu.VMEM((_BPW,), f32), pltpu.VMEM((_BPW,), f32),
        pltpu.VMEM((_BPW,), f32), pltpu.VMEM((_BPW,), f32),
        pltpu.VMEM((_BPW,), i32), pltpu.VMEM((_BPW,), i32),
        pltpu.VMEM((_BPW,), i32), pltpu.VMEM((_BPW,), i32),
        pltpu.VMEM((_BPW,), i32), pltpu.VMEM((_BPW,), i32),
        pltpu.VMEM((16, 128), i32),
        pltpu.VMEM((27, 128), i32),
        pltpu.VMEM((64 * _BPW,), f32),
        pltpu.VMEM((2, 128, 128), f32),
        pltpu.VMEM((216, _BPW), f32),
        pltpu.SemaphoreType.DMA, pltpu.SemaphoreType.DMA,
    ]
    mesh = plsc.VectorSubcoreMesh(core_axis_name="c", subcore_axis_name="s",
                                  num_cores=_NC, num_subcores=_NS)
    fn = pl.kernel(_sc_body, out_type, mesh=mesh, scratch_types=scratch,
                   compiler_params=pltpu.CompilerParams(
                       use_tc_tiling_on_sc=False, needs_layout_passes=False),
                   interpret=_INTERPRET)
    return fn(px, py, pz, t, t0f, ts2t)


def _a2_body(lin2_ref, lin1_ref, lin3_ref, srow_ref,
             t2_ref, t1_ref, ts1_ref, wd_ref,
             h2_ref, tsraw_ref, f2b, f1b, tsb, w2_v, w1_v, sem, wsem):
    rb = f2b.shape[1]
    b = pl.program_id(0)
    nb = pl.num_programs(0)

    def wcps():
        return [
            pltpu.make_async_copy(wd_ref.at[pl.ds(576, 4096), :], w2_v, wsem),
            pltpu.make_async_copy(wd_ref.at[pl.ds(64, 512), :], w1_v, wsem),
        ]

    @pl.when(b == 0)
    def _():
        for cp in wcps():
            cp.start()

    def cps(bb, ib):
        out = []
        for i in range(rb):
            r2 = lin2_ref[bb * rb + i]
            r1 = lin1_ref[bb * rb + i]
            c3 = lin3_ref[bb * rb + i]
            sr = srow_ref[bb * rb + i]
            out.append(pltpu.make_async_copy(
                t2_ref.at[pl.ds(r2, 1), :], f2b.at[ib, pl.ds(i, 1), :], sem))
            out.append(pltpu.make_async_copy(
                t1_ref.at[pl.ds(r1, 1), :], f1b.at[ib, pl.ds(i, 1), :], sem))
            out.append(pltpu.make_async_copy(
                ts1_ref.at[pl.ds(c3, 1), pl.ds(sr, 3), :],
                tsb.at[ib, pl.ds(i, 1), pl.ds(0, 3), :], sem))
            out.append(pltpu.make_async_copy(
                ts1_ref.at[pl.ds(c3, 1), pl.ds(63, 1), :],
                tsb.at[ib, pl.ds(i, 1), pl.ds(3, 1), :], sem))
        return out

    @pl.when(b == 0)
    def _():
        for cp in cps(b, 0):
            cp.start()

    @pl.when(b + 1 < nb)
    def _():
        for cp in cps(b + 1, (b + 1) % 2):
            cp.start()

    for cp in cps(b, b % 2):
        cp.wait()

    @pl.when(b == 0)
    def _():
        for cp in wcps():
            cp.wait()

    def compute(ib):
        h2_ref[...] = (
            jnp.dot(jnp.maximum(f2b[ib], 0.0), w2_v[...],
                    preferred_element_type=jnp.float32)
            + jnp.dot(jnp.maximum(f1b[ib], 0.0), w1_v[...],
                      preferred_element_type=jnp.float32))
        tsraw_ref[...] = tsb[ib]

    @pl.when(b % 2 == 0)
    def _():
        compute(0)

    @pl.when(b % 2 == 1)
    def _():
        compute(1)


def _a2_call(lin2, lin1, lin3, srow, t2r, t1r, ts1c, w_down, rb=64):
    nb = _B // rb
    grid_spec = pltpu.PrefetchScalarGridSpec(
        num_scalar_prefetch=4,
        grid=(nb,),
        in_specs=[
            pl.BlockSpec(memory_space=pltpu.MemorySpace.HBM),
            pl.BlockSpec(memory_space=pltpu.MemorySpace.HBM),
            pl.BlockSpec(memory_space=pltpu.MemorySpace.HBM),
            pl.BlockSpec(memory_space=pltpu.MemorySpace.HBM),
        ],
        out_specs=[
            pl.BlockSpec((rb, 128), lambda b, *_: (b, 0)),
            pl.BlockSpec((rb, 4, 64), lambda b, *_: (b, 0, 0)),
        ],
        scratch_shapes=[
            pltpu.VMEM((2, rb, 4096), jnp.float32),
            pltpu.VMEM((2, rb, 512), jnp.float32),
            pltpu.VMEM((2, rb, 4, 64), jnp.float32),
            pltpu.VMEM((4096, 128), jnp.float32),
            pltpu.VMEM((512, 128), jnp.float32),
            pltpu.SemaphoreType.DMA,
            pltpu.SemaphoreType.DMA,
        ],
    )
    fn = pl.pallas_call(
        _a2_body,
        grid_spec=grid_spec,
        out_shape=[
            jax.ShapeDtypeStruct((_B, 128), jnp.float32),
            jax.ShapeDtypeStruct((_B, 4, 64), jnp.float32),
        ],
        interpret=_INTERPRET,
    )
    return fn(lin2, lin1, lin3, srow, t2r, t1r, ts1c, w_down)


def _gmm_body(nk, nb, rb, kc, woff, lin_ref, tbl_ref, wd_ref, out_ref,
              buf, w_v, sem, wsem):
    k = pl.program_id(0)
    b = pl.program_id(1)
    s = k * nb + b

    def cps(kk, bb, ib):
        out = []
        for i in range(rb):
            row = lin_ref[bb * rb + i]
            out.append(pltpu.make_async_copy(
                tbl_ref.at[pl.ds(row, 1), pl.ds(kk * kc, kc)],
                buf.at[ib, pl.ds(i, 1), :], sem))
        return out

    @pl.when(b == 0)
    def _():
        pltpu.make_async_copy(
            wd_ref.at[pl.ds(woff + k * kc, kc), :], w_v, wsem).start()

    @pl.when(s == 0)
    def _():
        for cp in cps(k, b, 0):
            cp.start()

    @pl.when(s + 1 < nk * nb)
    def _():
        sn = s + 1
        for cp in cps(sn // nb, sn % nb, sn % 2):
            cp.start()

    for cp in cps(k, b, s % 2):
        cp.wait()

    @pl.when(b == 0)
    def _():
        pltpu.make_async_copy(
            wd_ref.at[pl.ds(woff + k * kc, kc), :], w_v, wsem).wait()

    w = w_v[...]

    @pl.when(s % 2 == 0)
    def _():
        out_ref[0] = jnp.dot(jnp.maximum(buf[0], 0.0), w,
                             preferred_element_type=jnp.float32)

    @pl.when(s % 2 == 1)
    def _():
        out_ref[0] = jnp.dot(jnp.maximum(buf[1], 0.0), w,
                             preferred_element_type=jnp.float32)


def _gather_matmul(lin, tbl2d, w_down, woff, nk, kc, rb):
    nb = _B // rb
    grid_spec = pltpu.PrefetchScalarGridSpec(
        num_scalar_prefetch=1,
        grid=(nk, nb),
        in_specs=[
            pl.BlockSpec(memory_space=pltpu.MemorySpace.HBM),
            pl.BlockSpec(memory_space=pltpu.MemorySpace.HBM),
        ],
        out_specs=pl.BlockSpec((1, rb, 128), lambda k, b, lin_: (k, b, 0)),
        scratch_shapes=[
            pltpu.VMEM((2, rb, kc), jnp.float32),
            pltpu.VMEM((kc, 128), jnp.float32),
            pltpu.SemaphoreType.DMA,
            pltpu.SemaphoreType.DMA,
        ],
    )
    fn = pl.pallas_call(
        functools.partial(_gmm_body, nk, nb, rb, kc, woff),
        grid_spec=grid_spec,
        out_shape=jax.ShapeDtypeStruct((nk, _B, 128), jnp.float32),
        compiler_params=pltpu.CompilerParams(
            vmem_limit_bytes=56 * 1024 * 1024),
        interpret=_INTERPRET,
    )
    return fn(lin, tbl2d, w_down)


def _tail_body(h3p_ref, h2_ref, f0T_ref, nfT_ref, tsraw_ref,
               w0_ref, wtm_ref, wt0_ref, wtp_ref, wnf_ref,
               dir_ref, t_ref, wmid_ref, lng_ref, lnb_ref, wfin_ref,
               bfin_ref, out_ref):
    f32 = jnp.float32

    def rmm(x, w_ref):
        return jnp.dot(jnp.maximum(x, 0.0), w_ref[...],
                       preferred_element_type=f32)

    def rtmm(xT_ref, w_ref):
        return lax.dot_general(jnp.maximum(xT_ref[...], 0.0), w_ref[...],
                               (((0,), (0,)), ((), ())),
                               preferred_element_type=f32)

    t64 = (t_ref[...] * 63.0).astype(jnp.int32)
    wrap = t64 == 0
    raw0 = tsraw_ref[:, 0, :]
    raw1 = tsraw_ref[:, 1, :]
    raw2 = tsraw_ref[:, 2, :]
    raw3 = tsraw_ref[:, 3, :]
    tfm = jnp.where(wrap, raw3, raw0)
    tf0 = jnp.where(wrap, raw0, raw1)
    tfp = jnp.where(wrap, raw1, raw2)

    h3 = h3p_ref[0]
    for kk in range(1, h3p_ref.shape[0]):
        h3 = h3 + h3p_ref[kk]
    h = (h3 + h2_ref[...]
         + rtmm(f0T_ref, w0_ref) + rtmm(nfT_ref, wnf_ref)
         + rmm(tfm, wtm_ref) + rmm(tf0, wt0_ref) + rmm(tfp, wtp_ref))
    wmid = wmid_ref[...]
    ff = (jnp.dot(h, wmid[0:128, :], preferred_element_type=f32)
          + jnp.dot(dir_ref[...], wmid[128:131, :], preferred_element_type=f32)
          + t_ref[...] * wmid[131, :][None, :])
    mu = jnp.mean(ff, axis=-1, keepdims=True)
    d = ff - mu
    var = jnp.mean(d * d, axis=-1, keepdims=True)
    ffn = d * lax.rsqrt(var + 1e-5) * lng_ref[...] + lnb_ref[...]
    out_ref[...] = (jnp.dot(ffn, wfin_ref[...], preferred_element_type=f32)
                    + bfin_ref[...])


def _tail(h3p, h2, f0T, nfT, tsraw, w0, wtm, wt0, wtp, wnf,
          dirs, t, wmid, lng, lnb, wfin, bfin):
    fn = pl.pallas_call(
        _tail_body,
        out_shape=jax.ShapeDtypeStruct((_B, 4), jnp.float32),
        interpret=_INTERPRET,
    )
    return fn(h3p, h2, f0T, nfT, tsraw, w0, wtm, wt0, wtp, wnf,
              dirs, t, wmid, lng, lnb, wfin, bfin)


def kernel(pos, dir, t, table0, table1, table2, table3, ts_table1, ts_table2,
           W_down, W_mid, ln_g, ln_b, W_fin, b_fin):
    px = pos[:, 0]
    py = pos[:, 1]
    pz = pos[:, 2]
    t0f = table0.transpose(0, 1, 3, 2).reshape(-1)
    ts2t = ts_table2.transpose(0, 1, 2, 4, 3).reshape(128 * 128 * 128 * 4, 128)
    t1r = table1.reshape(64 * 64 * 64, 512)
    t2r = table2.reshape(32 * 32 * 32, 4096)
    t3r = table3.reshape(16 * 16 * 16, 32768)
    ts1c = ts_table1.reshape(16 * 16 * 16, 64, 64)

    f0g, nfT, lin1, lin2, lin3, srow = _sc_gather(px, py, pz, t, t0f, ts2t)
    f0T = f0g.reshape(_NW, 64, _BPW).transpose(1, 0, 2).reshape(64, _B)

    h2, tsraw = _a2_call(lin2, lin1, lin3, srow, t2r, t1r, ts1c, W_down)
    h3p = _gather_matmul(lin3, t3r, W_down, woff=4672, nk=1, kc=32768, rb=64)

    return _tail(
        h3p, h2, f0T, nfT, tsraw,
        W_down[0:64],
        W_down[37440:37504], W_down[37504:37568], W_down[37568:37632],
        W_down[37632:37848],
        dir, t.reshape(_B, 1), W_mid, ln_g.reshape(1, 132),
        ln_b.reshape(1, 132), W_fin, b_fin.reshape(1, 4))


# speedup vs baseline: 16.1235x; 1.0110x over previous
import functools

import jax
import jax.numpy as jnp
from jax import lax
from jax.experimental import pallas as pl
from jax.experimental.pallas import tpu as pltpu
from jax.experimental.pallas import tpu_sc as plsc

_B = 1024
_NC, _NS, _LANES = 2, 16, 16
_NW = _NC * _NS
_BPW = _B // _NW

_INTERPRET = False

_SPATIAL_OFFS = [(dx, dy, dz) for dx in (-1, 0, 1) for dy in (-1, 0, 1)
                 for dz in (-1, 0, 1)]


def _sc_body(px_h, py_h, pz_h, t_h, t0f_h, ts2t_h,
             f0g_h, nfT_h, lin1_h, lin2_h, lin3_h, srow_h,
             px_v, py_v, pz_v, t_v,
             lin1_v, lin2_v, lin3_v, srow_v, tm_v, tp_v,
             widx_v, nbrg_v, f0g_v, nfw_v, nfT_v, sem, sem2):
    wid = lax.axis_index("s") * _NC + lax.axis_index("c")
    base = wid * _BPW

    pltpu.sync_copy(px_h.at[pl.ds(base, _BPW)], px_v)
    pltpu.sync_copy(py_h.at[pl.ds(base, _BPW)], py_v)
    pltpu.sync_copy(pz_h.at[pl.ds(base, _BPW)], pz_v)
    pltpu.sync_copy(t_h.at[pl.ds(base, _BPW)], t_v)

    for c in range(_BPW // _LANES):
        sl = pl.ds(c * _LANES, _LANES)
        px = px_v[sl]
        py = py_v[sl]
        pz = pz_v[sl]
        tt = t_v[sl]
        x0 = (px * 127.0).astype(jnp.int32)
        y0 = (py * 127.0).astype(jnp.int32)
        z0 = (pz * 127.0).astype(jnp.int32)
        x1 = (px * 63.0).astype(jnp.int32)
        y1 = (py * 63.0).astype(jnp.int32)
        z1 = (pz * 63.0).astype(jnp.int32)
        x2 = (px * 31.0).astype(jnp.int32)
        y2 = (py * 31.0).astype(jnp.int32)
        z2 = (pz * 31.0).astype(jnp.int32)
        x3 = (px * 15.0).astype(jnp.int32)
        y3 = (py * 15.0).astype(jnp.int32)
        z3 = (pz * 15.0).astype(jnp.int32)
        t127 = (tt * 127.0).astype(jnp.int32)
        t64 = (tt * 63.0).astype(jnp.int32)

        lin1_v[sl] = (x1 * 64 + y1) * 64 + z1
        lin2_v[sl] = (x2 * 32 + y2) * 32 + z2
        lin3 = (x3 * 16 + y3) * 16 + z3
        lin3_v[sl] = lin3
        srow_v[sl] = jnp.minimum(jnp.maximum(t64 - 1, 0), 61)
        tm_v[sl] = (t127 + 127) & 127
        tp_v[sl] = (t127 + 129) & 127

        base0 = (x0 * 128 + y0) * 8192 + z0
        for f in range(64):
            p = f * _BPW + c * _LANES
            widx_v[p >> 7, pl.ds(p & 127, _LANES)] = base0 + f * 128

        xs = {-1: (x3 + 127) & 127, 0: x3, 1: (x3 + 129) & 127}
        ys = {-1: (y3 + 127) & 127, 0: y3, 1: (y3 + 129) & 127}
        zs = {-1: (z3 + 127) & 127, 0: z3, 1: (z3 + 129) & 127}
        for s, (dx, dy, dz) in enumerate(_SPATIAL_OFFS):
            spb = ((xs[dx] * 128 + ys[dy]) * 128 + zs[dz]) * 4
            for f in range(4):
                nbrg_v[s, pl.ds(f * _BPW + c * _LANES, _LANES)] = spb + f

    ocps = [
        pltpu.async_copy(lin1_v, lin1_h.at[pl.ds(base, _BPW)], sem2),
        pltpu.async_copy(lin2_v, lin2_h.at[pl.ds(base, _BPW)], sem2),
        pltpu.async_copy(lin3_v, lin3_h.at[pl.ds(base, _BPW)], sem2),
        pltpu.async_copy(srow_v, srow_h.at[pl.ds(base, _BPW)], sem2),
    ]

    iot = lax.iota(jnp.int32, _LANES)
    f0cps = [pltpu.async_copy(t0f_h.at[widx_v.at[r]],
                              f0g_v.at[pl.ds(r * 128, 128)], sem2)
             for r in range(16)]

    _D = 5

    def nf_fire(s):
        return pltpu.async_copy(ts2t_h.at[nbrg_v.at[s]], nfw_v.at[s % _D],
                                sem)

    def nf_extract(s):
        ib = s % _D
        for c in range(_BPW // _LANES):
            tm = tm_v[pl.ds(c * _LANES, _LANES)]
            tp = tp_v[pl.ds(c * _LANES, _LANES)]
            for f in range(4):
                rows = f * _BPW + c * _LANES + iot
                vm = plsc.load_gather(nfw_v.at[ib], [rows, tm])
                vp = plsc.load_gather(nfw_v.at[ib], [rows, tp])
                nfT_v[(s * 2 + 0) * 4 + f, pl.ds(c * _LANES, _LANES)] = vm
                nfT_v[(s * 2 + 1) * 4 + f, pl.ds(c * _LANES, _LANES)] = vp

    cps = {}
    for s in range(_D - 1):
        cps[s] = nf_fire(s)
    for s in range(27):
        if s + _D - 1 < 27:
            cps[s + _D - 1] = nf_fire(s + _D - 1)
        cps[s].wait()
        nf_extract(s)

    for cp in f0cps:
        cp.wait()
    for cp in ocps:
        cp.wait()

    pltpu.sync_copy(f0g_v, f0g_h.at[wid])
    pltpu.sync_copy(nfT_v, nfT_h.at[:, pl.ds(base, _BPW)])


def _sc_gather(px, py, pz, t, t0f, ts2t):
    f32, i32 = jnp.float32, jnp.int32
    out_type = (
        jax.ShapeDtypeStruct((_NW, 64 * _BPW), f32),
        jax.ShapeDtypeStruct((216, _B), f32),
        jax.ShapeDtypeStruct((_B,), i32),
        jax.ShapeDtypeStruct((_B,), i32),
        jax.ShapeDtypeStruct((_B,), i32),
        jax.ShapeDtypeStruct((_B,), i32),
    )
    scratch = [
        pltpu.VMEM((_BPW,), f32), pltpu.VMEM((_BPW,), f32),
        pltpu.VMEM((_BPW,), f32), pltpu.VMEM((_BPW,), f32),
        pltpu.VMEM((_BPW,), i32), pltpu.VMEM((_BPW,), i32),
        pltpu.VMEM((_BPW,), i32), pltpu.VMEM((_BPW,), i32),
        pltpu.VMEM((_BPW,), i32), pltpu.VMEM((_BPW,), i32),
        pltpu.VMEM((16, 128), i32),
        pltpu.VMEM((27, 128), i32),
        pltpu.VMEM((64 * _BPW,), f32),
        pltpu.VMEM((5, 128, 128), f32),
        pltpu.VMEM((216, _BPW), f32),
        pltpu.SemaphoreType.DMA, pltpu.SemaphoreType.DMA,
    ]
    mesh = plsc.VectorSubcoreMesh(core_axis_name="c", subcore_axis_name="s",
                                  num_cores=_NC, num_subcores=_NS)
    fn = pl.kernel(_sc_body, out_type, mesh=mesh, scratch_types=scratch,
                   compiler_params=pltpu.CompilerParams(
                       use_tc_tiling_on_sc=False, needs_layout_passes=False),
                   interpret=_INTERPRET)
    return fn(px, py, pz, t, t0f, ts2t)


def _a2_body(lin2_ref, lin1_ref, lin3_ref, srow_ref,
             t2_ref, t1_ref, ts1_ref, wd_ref,
             h2_ref, tsraw_ref, f2b, f1b, tsb, w2_v, w1_v, sem, wsem):
    rb = f2b.shape[1]
    b = pl.program_id(0)
    nb = pl.num_programs(0)

    def wcps():
        return [
            pltpu.make_async_copy(wd_ref.at[pl.ds(576, 4096), :], w2_v, wsem),
            pltpu.make_async_copy(wd_ref.at[pl.ds(64, 512), :], w1_v, wsem),
        ]

    @pl.when(b == 0)
    def _():
        for cp in wcps():
            cp.start()

    def cps(bb, ib):
        out = []
        for i in range(rb):
            r2 = lin2_ref[bb * rb + i]
            r1 = lin1_ref[bb * rb + i]
            c3 = lin3_ref[bb * rb + i]
            sr = srow_ref[bb * rb + i]
            out.append(pltpu.make_async_copy(
                t2_ref.at[pl.ds(r2, 1), :], f2b.at[ib, pl.ds(i, 1), :], sem))
            out.append(pltpu.make_async_copy(
                t1_ref.at[pl.ds(r1, 1), :], f1b.at[ib, pl.ds(i, 1), :], sem))
            out.append(pltpu.make_async_copy(
                ts1_ref.at[pl.ds(c3, 1), pl.ds(sr, 3), :],
                tsb.at[ib, pl.ds(i, 1), pl.ds(0, 3), :], sem))
            out.append(pltpu.make_async_copy(
                ts1_ref.at[pl.ds(c3, 1), pl.ds(63, 1), :],
                tsb.at[ib, pl.ds(i, 1), pl.ds(3, 1), :], sem))
        return out

    @pl.when(b == 0)
    def _():
        for cp in cps(b, 0):
            cp.start()

    @pl.when(b + 1 < nb)
    def _():
        for cp in cps(b + 1, (b + 1) % 2):
            cp.start()

    def drain(ib):
        pltpu.make_async_copy(
            t2_ref.at[pl.ds(0, rb), :], f2b.at[ib], sem).wait()
        pltpu.make_async_copy(
            t1_ref.at[pl.ds(0, rb), :], f1b.at[ib], sem).wait()
        pltpu.make_async_copy(
            ts1_ref.at[pl.ds(0, rb), pl.ds(0, 4), :], tsb.at[ib], sem).wait()

    @pl.when(b % 2 == 0)
    def _():
        drain(0)

    @pl.when(b % 2 == 1)
    def _():
        drain(1)

    @pl.when(b == 0)
    def _():
        for cp in wcps():
            cp.wait()

    def compute(ib):
        h2_ref[...] = (
            jnp.dot(jnp.maximum(f2b[ib], 0.0), w2_v[...],
                    preferred_element_type=jnp.float32)
            + jnp.dot(jnp.maximum(f1b[ib], 0.0), w1_v[...],
                      preferred_element_type=jnp.float32))
        tsraw_ref[...] = tsb[ib]

    @pl.when(b % 2 == 0)
    def _():
        compute(0)

    @pl.when(b % 2 == 1)
    def _():
        compute(1)


def _a2_call(lin2, lin1, lin3, srow, t2r, t1r, ts1c, w_down, rb=64):
    nb = _B // rb
    grid_spec = pltpu.PrefetchScalarGridSpec(
        num_scalar_prefetch=4,
        grid=(nb,),
        in_specs=[
            pl.BlockSpec(memory_space=pltpu.MemorySpace.HBM),
            pl.BlockSpec(memory_space=pltpu.MemorySpace.HBM),
            pl.BlockSpec(memory_space=pltpu.MemorySpace.HBM),
            pl.BlockSpec(memory_space=pltpu.MemorySpace.HBM),
        ],
        out_specs=[
            pl.BlockSpec((rb, 128), lambda b, *_: (b, 0)),
            pl.BlockSpec((rb, 4, 64), lambda b, *_: (b, 0, 0)),
        ],
        scratch_shapes=[
            pltpu.VMEM((2, rb, 4096), jnp.float32),
            pltpu.VMEM((2, rb, 512), jnp.float32),
            pltpu.VMEM((2, rb, 4, 64), jnp.float32),
            pltpu.VMEM((4096, 128), jnp.float32),
            pltpu.VMEM((512, 128), jnp.float32),
            pltpu.SemaphoreType.DMA,
            pltpu.SemaphoreType.DMA,
        ],
    )
    fn = pl.pallas_call(
        _a2_body,
        grid_spec=grid_spec,
        out_shape=[
            jax.ShapeDtypeStruct((_B, 128), jnp.float32),
            jax.ShapeDtypeStruct((_B, 4, 64), jnp.float32),
        ],
        interpret=_INTERPRET,
    )
    return fn(lin2, lin1, lin3, srow, t2r, t1r, ts1c, w_down)


def _gmm_body(nk, nb, rb, kc, woff, lin_ref, tbl_ref, wd_ref, out_ref,
              buf, w_v, sem, wsem):
    k = pl.program_id(0)
    b = pl.program_id(1)
    s = k * nb + b

    def cps(kk, bb, ib):
        out = []
        for i in range(rb):
            row = lin_ref[bb * rb + i]
            out.append(pltpu.make_async_copy(
                tbl_ref.at[pl.ds(row, 1), pl.ds(kk * kc, kc)],
                buf.at[ib, pl.ds(i, 1), :], sem))
        return out

    @pl.when(b == 0)
    def _():
        pltpu.make_async_copy(
            wd_ref.at[pl.ds(woff + k * kc, kc), :], w_v, wsem).start()

    @pl.when(s == 0)
    def _():
        for cp in cps(k, b, 0):
            cp.start()

    @pl.when(s + 1 < nk * nb)
    def _():
        sn = s + 1
        for cp in cps(sn // nb, sn % nb, sn % 2):
            cp.start()

    def drain(ib):
        pltpu.make_async_copy(
            tbl_ref.at[pl.ds(0, rb), pl.ds(0, kc)], buf.at[ib], sem).wait()

    @pl.when(s % 2 == 0)
    def _():
        drain(0)

    @pl.when(s % 2 == 1)
    def _():
        drain(1)

    @pl.when(b == 0)
    def _():
        pltpu.make_async_copy(
            wd_ref.at[pl.ds(woff + k * kc, kc), :], w_v, wsem).wait()

    w = w_v[...]

    @pl.when(s % 2 == 0)
    def _():
        out_ref[0] = jnp.dot(jnp.maximum(buf[0], 0.0), w,
                             preferred_element_type=jnp.float32)

    @pl.when(s % 2 == 1)
    def _():
        out_ref[0] = jnp.dot(jnp.maximum(buf[1], 0.0), w,
                             preferred_element_type=jnp.float32)


def _gather_matmul(lin, tbl2d, w_down, woff, nk, kc, rb):
    nb = _B // rb
    grid_spec = pltpu.PrefetchScalarGridSpec(
        num_scalar_prefetch=1,
        grid=(nk, nb),
        in_specs=[
            pl.BlockSpec(memory_space=pltpu.MemorySpace.HBM),
            pl.BlockSpec(memory_space=pltpu.MemorySpace.HBM),
        ],
        out_specs=pl.BlockSpec((1, rb, 128), lambda k, b, lin_: (k, b, 0)),
        scratch_shapes=[
            pltpu.VMEM((2, rb, kc), jnp.float32),
            pltpu.VMEM((kc, 128), jnp.float32),
            pltpu.SemaphoreType.DMA,
            pltpu.SemaphoreType.DMA,
        ],
    )
    fn = pl.pallas_call(
        functools.partial(_gmm_body, nk, nb, rb, kc, woff),
        grid_spec=grid_spec,
        out_shape=jax.ShapeDtypeStruct((nk, _B, 128), jnp.float32),
        compiler_params=pltpu.CompilerParams(
            vmem_limit_bytes=56 * 1024 * 1024),
        interpret=_INTERPRET,
    )
    return fn(lin, tbl2d, w_down)


def _tail_body(h3p_ref, h2_ref, f0T_ref, nfT_ref, tsraw_ref,
               w0_ref, wtm_ref, wt0_ref, wtp_ref, wnf_ref,
               dir_ref, t_ref, wmid_ref, lng_ref, lnb_ref, wfin_ref,
               bfin_ref, out_ref):
    f32 = jnp.float32

    def rmm(x, w_ref):
        return jnp.dot(jnp.maximum(x, 0.0), w_ref[...],
                       preferred_element_type=f32)

    def rtmm(xT_ref, w_ref):
        return lax.dot_general(jnp.maximum(xT_ref[...], 0.0), w_ref[...],
                               (((0,), (0,)), ((), ())),
                               preferred_element_type=f32)

    t64 = (t_ref[...] * 63.0).astype(jnp.int32)
    wrap = t64 == 0
    raw0 = tsraw_ref[:, 0, :]
    raw1 = tsraw_ref[:, 1, :]
    raw2 = tsraw_ref[:, 2, :]
    raw3 = tsraw_ref[:, 3, :]
    tfm = jnp.where(wrap, raw3, raw0)
    tf0 = jnp.where(wrap, raw0, raw1)
    tfp = jnp.where(wrap, raw1, raw2)

    h3 = h3p_ref[0]
    for kk in range(1, h3p_ref.shape[0]):
        h3 = h3 + h3p_ref[kk]
    h = (h3 + h2_ref[...]
         + rtmm(f0T_ref, w0_ref) + rtmm(nfT_ref, wnf_ref)
         + rmm(tfm, wtm_ref) + rmm(tf0, wt0_ref) + rmm(tfp, wtp_ref))
    wmid = wmid_ref[...]
    ff = (jnp.dot(h, wmid[0:128, :], preferred_element_type=f32)
          + jnp.dot(dir_ref[...], wmid[128:131, :], preferred_element_type=f32)
          + t_ref[...] * wmid[131, :][None, :])
    mu = jnp.mean(ff, axis=-1, keepdims=True)
    d = ff - mu
    var = jnp.mean(d * d, axis=-1, keepdims=True)
    ffn = d * lax.rsqrt(var + 1e-5) * lng_ref[...] + lnb_ref[...]
    out_ref[...] = (jnp.dot(ffn, wfin_ref[...], preferred_element_type=f32)
                    + bfin_ref[...])


def _tail(h3p, h2, f0T, nfT, tsraw, w0, wtm, wt0, wtp, wnf,
          dirs, t, wmid, lng, lnb, wfin, bfin):
    fn = pl.pallas_call(
        _tail_body,
        out_shape=jax.ShapeDtypeStruct((_B, 4), jnp.float32),
        interpret=_INTERPRET,
    )
    return fn(h3p, h2, f0T, nfT, tsraw, w0, wtm, wt0, wtp, wnf,
              dirs, t, wmid, lng, lnb, wfin, bfin)


def kernel(pos, dir, t, table0, table1, table2, table3, ts_table1, ts_table2,
           W_down, W_mid, ln_g, ln_b, W_fin, b_fin):
    px = pos[:, 0]
    py = pos[:, 1]
    pz = pos[:, 2]
    t0f = table0.transpose(0, 1, 3, 2).reshape(-1)
    ts2t = ts_table2.transpose(0, 1, 2, 4, 3).reshape(128 * 128 * 128 * 4, 128)
    t1r = table1.reshape(64 * 64 * 64, 512)
    t2r = table2.reshape(32 * 32 * 32, 4096)
    t3r = table3.reshape(16 * 16 * 16, 32768)
    ts1c = ts_table1.reshape(16 * 16 * 16, 64, 64)

    f0g, nfT, lin1, lin2, lin3, srow = _sc_gather(px, py, pz, t, t0f, ts2t)
    f0T = f0g.reshape(_NW, 64, _BPW).transpose(1, 0, 2).reshape(64, _B)

    h2, tsraw = _a2_call(lin2, lin1, lin3, srow, t2r, t1r, ts1c, W_down)
    h3p = _gather_matmul(lin3, t3r, W_down, woff=4672, nk=1, kc=32768, rb=64)

    return _tail(
        h3p, h2, f0T, nfT, tsraw,
        W_down[0:64],
        W_down[37440:37504], W_down[37504:37568], W_down[37568:37632],
        W_down[37632:37848],
        dir, t.reshape(_B, 1), W_mid, ln_g.reshape(1, 132),
        ln_b.reshape(1, 132), W_fin, b_fin.reshape(1, 4))


# speedup vs baseline: 16.8314x; 1.0439x over previous
import functools

import jax
import jax.numpy as jnp
from jax import lax
from jax.experimental import pallas as pl
from jax.experimental.pallas import tpu as pltpu
from jax.experimental.pallas import tpu_sc as plsc

_B = 1024
_NC, _NS, _LANES = 2, 16, 16
_NW = _NC * _NS
_BPW = _B // _NW

_INTERPRET = False

_SPATIAL_OFFS = [(dx, dy, dz) for dx in (-1, 0, 1) for dy in (-1, 0, 1)
                 for dz in (-1, 0, 1)]


def _sc_body(px_h, py_h, pz_h, t_h, t0f_h, ts2t_h,
             f0g_h, nfT_h,
             px_v, py_v, pz_v, t_v, tm_v, tp_v,
             widx_v, nbrg_v, f0g_v, nfw_v, nfT_v, sem, sem2):
    wid = lax.axis_index("s") * _NC + lax.axis_index("c")
    base = wid * _BPW

    pltpu.sync_copy(px_h.at[pl.ds(base, _BPW)], px_v)
    pltpu.sync_copy(py_h.at[pl.ds(base, _BPW)], py_v)
    pltpu.sync_copy(pz_h.at[pl.ds(base, _BPW)], pz_v)
    pltpu.sync_copy(t_h.at[pl.ds(base, _BPW)], t_v)

    for c in range(_BPW // _LANES):
        sl = pl.ds(c * _LANES, _LANES)
        px = px_v[sl]
        py = py_v[sl]
        pz = pz_v[sl]
        tt = t_v[sl]
        x0 = (px * 127.0).astype(jnp.int32)
        y0 = (py * 127.0).astype(jnp.int32)
        z0 = (pz * 127.0).astype(jnp.int32)
        x3 = (px * 15.0).astype(jnp.int32)
        y3 = (py * 15.0).astype(jnp.int32)
        z3 = (pz * 15.0).astype(jnp.int32)
        t127 = (tt * 127.0).astype(jnp.int32)
        tm_v[sl] = (t127 + 127) & 127
        tp_v[sl] = (t127 + 129) & 127

        base0 = (x0 * 128 + y0) * 8192 + z0
        for f in range(64):
            p = f * _BPW + c * _LANES
            widx_v[p >> 7, pl.ds(p & 127, _LANES)] = base0 + f * 128

        xs = {-1: (x3 + 127) & 127, 0: x3, 1: (x3 + 129) & 127}
        ys = {-1: (y3 + 127) & 127, 0: y3, 1: (y3 + 129) & 127}
        zs = {-1: (z3 + 127) & 127, 0: z3, 1: (z3 + 129) & 127}
        for s, (dx, dy, dz) in enumerate(_SPATIAL_OFFS):
            spb = ((xs[dx] * 128 + ys[dy]) * 128 + zs[dz]) * 4
            for f in range(4):
                nbrg_v[s, pl.ds(f * _BPW + c * _LANES, _LANES)] = spb + f

    iot = lax.iota(jnp.int32, _LANES)
    f0cps = [pltpu.async_copy(t0f_h.at[widx_v.at[r]],
                              f0g_v.at[pl.ds(r * 128, 128)], sem2)
             for r in range(16)]

    _D = 5

    def nf_fire(s):
        return pltpu.async_copy(ts2t_h.at[nbrg_v.at[s]], nfw_v.at[s % _D],
                                sem)

    def nf_extract(s):
        ib = s % _D
        for c in range(_BPW // _LANES):
            tm = tm_v[pl.ds(c * _LANES, _LANES)]
            tp = tp_v[pl.ds(c * _LANES, _LANES)]
            for f in range(4):
                rows = f * _BPW + c * _LANES + iot
                vm = plsc.load_gather(nfw_v.at[ib], [rows, tm])
                vp = plsc.load_gather(nfw_v.at[ib], [rows, tp])
                nfT_v[(s * 2 + 0) * 4 + f, pl.ds(c * _LANES, _LANES)] = vm
                nfT_v[(s * 2 + 1) * 4 + f, pl.ds(c * _LANES, _LANES)] = vp

    cps = {}
    for s in range(_D - 1):
        cps[s] = nf_fire(s)
    for s in range(27):
        if s + _D - 1 < 27:
            cps[s + _D - 1] = nf_fire(s + _D - 1)
        cps[s].wait()
        nf_extract(s)

    for cp in f0cps:
        cp.wait()

    pltpu.sync_copy(f0g_v, f0g_h.at[wid])
    pltpu.sync_copy(nfT_v, nfT_h.at[:, pl.ds(base, _BPW)])


def _sc_gather(px, py, pz, t, t0f, ts2t):
    f32, i32 = jnp.float32, jnp.int32
    out_type = (
        jax.ShapeDtypeStruct((_NW, 64 * _BPW), f32),
        jax.ShapeDtypeStruct((216, _B), f32),
    )
    scratch = [
        pltpu.VMEM((_BPW,), f32), pltpu.VMEM((_BPW,), f32),
        pltpu.VMEM((_BPW,), f32), pltpu.VMEM((_BPW,), f32),
        pltpu.VMEM((_BPW,), i32), pltpu.VMEM((_BPW,), i32),
        pltpu.VMEM((16, 128), i32),
        pltpu.VMEM((27, 128), i32),
        pltpu.VMEM((64 * _BPW,), f32),
        pltpu.VMEM((5, 128, 128), f32),
        pltpu.VMEM((216, _BPW), f32),
        pltpu.SemaphoreType.DMA, pltpu.SemaphoreType.DMA,
    ]
    mesh = plsc.VectorSubcoreMesh(core_axis_name="c", subcore_axis_name="s",
                                  num_cores=_NC, num_subcores=_NS)
    fn = pl.kernel(_sc_body, out_type, mesh=mesh, scratch_types=scratch,
                   compiler_params=pltpu.CompilerParams(
                       use_tc_tiling_on_sc=False, needs_layout_passes=False),
                   interpret=_INTERPRET)
    return fn(px, py, pz, t, t0f, ts2t)


def _idx_body(posT_ref, t_ref, out_ref):
    i32 = jnp.int32
    px = posT_ref[0:1, :]
    py = posT_ref[1:2, :]
    pz = posT_ref[2:3, :]
    x1 = (px * 63.0).astype(i32)
    y1 = (py * 63.0).astype(i32)
    z1 = (pz * 63.0).astype(i32)
    x2 = (px * 31.0).astype(i32)
    y2 = (py * 31.0).astype(i32)
    z2 = (pz * 31.0).astype(i32)
    x3 = (px * 15.0).astype(i32)
    y3 = (py * 15.0).astype(i32)
    z3 = (pz * 15.0).astype(i32)
    t64 = (t_ref[...] * 63.0).astype(i32)
    out_ref[0:1, :] = (x1 * 64 + y1) * 64 + z1
    out_ref[1:2, :] = (x2 * 32 + y2) * 32 + z2
    out_ref[2:3, :] = (x3 * 16 + y3) * 16 + z3
    out_ref[3:4, :] = jnp.minimum(jnp.maximum(t64 - 1, 0), 61)


def _idx_call(posT, t1):
    fn = pl.pallas_call(
        _idx_body,
        out_shape=jax.ShapeDtypeStruct((4, _B), jnp.int32),
        interpret=_INTERPRET,
    )
    return fn(posT, t1)


def _a2_body(lin2_ref, lin1_ref, lin3_ref, srow_ref,
             t2_ref, t1_ref, ts1_ref, wd_ref,
             h2_ref, tsraw_ref, f2b, f1b, tsb, w2_v, w1_v, sem, wsem):
    rb = f2b.shape[1]
    b = pl.program_id(0)
    nb = pl.num_programs(0)

    def wcps():
        return [
            pltpu.make_async_copy(wd_ref.at[pl.ds(576, 4096), :], w2_v, wsem),
            pltpu.make_async_copy(wd_ref.at[pl.ds(64, 512), :], w1_v, wsem),
        ]

    @pl.when(b == 0)
    def _():
        for cp in wcps():
            cp.start()

    def cps(bb, ib):
        out = []
        for i in range(rb):
            r2 = lin2_ref[bb * rb + i]
            r1 = lin1_ref[bb * rb + i]
            c3 = lin3_ref[bb * rb + i]
            sr = srow_ref[bb * rb + i]
            out.append(pltpu.make_async_copy(
                t2_ref.at[pl.ds(r2, 1), :], f2b.at[ib, pl.ds(i, 1), :], sem))
            out.append(pltpu.make_async_copy(
                t1_ref.at[pl.ds(r1, 1), :], f1b.at[ib, pl.ds(i, 1), :], sem))
            out.append(pltpu.make_async_copy(
                ts1_ref.at[pl.ds(c3, 1), pl.ds(sr, 3), :],
                tsb.at[ib, pl.ds(i, 1), pl.ds(0, 3), :], sem))
            out.append(pltpu.make_async_copy(
                ts1_ref.at[pl.ds(c3, 1), pl.ds(63, 1), :],
                tsb.at[ib, pl.ds(i, 1), pl.ds(3, 1), :], sem))
        return out

    @pl.when(b == 0)
    def _():
        for cp in cps(b, 0):
            cp.start()

    @pl.when(b + 1 < nb)
    def _():
        for cp in cps(b + 1, (b + 1) % 2):
            cp.start()

    def drain(ib):
        pltpu.make_async_copy(
            t2_ref.at[pl.ds(0, rb), :], f2b.at[ib], sem).wait()
        pltpu.make_async_copy(
            t1_ref.at[pl.ds(0, rb), :], f1b.at[ib], sem).wait()
        pltpu.make_async_copy(
            ts1_ref.at[pl.ds(0, rb), pl.ds(0, 4), :], tsb.at[ib], sem).wait()

    @pl.when(b % 2 == 0)
    def _():
        drain(0)

    @pl.when(b % 2 == 1)
    def _():
        drain(1)

    @pl.when(b == 0)
    def _():
        for cp in wcps():
            cp.wait()

    def compute(ib):
        h2_ref[...] = (
            jnp.dot(jnp.maximum(f2b[ib], 0.0), w2_v[...],
                    preferred_element_type=jnp.float32)
            + jnp.dot(jnp.maximum(f1b[ib], 0.0), w1_v[...],
                      preferred_element_type=jnp.float32))
        tsraw_ref[...] = tsb[ib]

    @pl.when(b % 2 == 0)
    def _():
        compute(0)

    @pl.when(b % 2 == 1)
    def _():
        compute(1)


def _a2_call(lin2, lin1, lin3, srow, t2r, t1r, ts1c, w_down, rb=64):
    nb = _B // rb
    grid_spec = pltpu.PrefetchScalarGridSpec(
        num_scalar_prefetch=4,
        grid=(nb,),
        in_specs=[
            pl.BlockSpec(memory_space=pltpu.MemorySpace.HBM),
            pl.BlockSpec(memory_space=pltpu.MemorySpace.HBM),
            pl.BlockSpec(memory_space=pltpu.MemorySpace.HBM),
            pl.BlockSpec(memory_space=pltpu.MemorySpace.HBM),
        ],
        out_specs=[
            pl.BlockSpec((rb, 128), lambda b, *_: (b, 0)),
            pl.BlockSpec((rb, 4, 64), lambda b, *_: (b, 0, 0)),
        ],
        scratch_shapes=[
            pltpu.VMEM((2, rb, 4096), jnp.float32),
            pltpu.VMEM((2, rb, 512), jnp.float32),
            pltpu.VMEM((2, rb, 4, 64), jnp.float32),
            pltpu.VMEM((4096, 128), jnp.float32),
            pltpu.VMEM((512, 128), jnp.float32),
            pltpu.SemaphoreType.DMA,
            pltpu.SemaphoreType.DMA,
        ],
    )
    fn = pl.pallas_call(
        _a2_body,
        grid_spec=grid_spec,
        out_shape=[
            jax.ShapeDtypeStruct((_B, 128), jnp.float32),
            jax.ShapeDtypeStruct((_B, 4, 64), jnp.float32),
        ],
        interpret=_INTERPRET,
    )
    return fn(lin2, lin1, lin3, srow, t2r, t1r, ts1c, w_down)


def _gmm_body(nk, nb, rb, kc, woff, lin_ref, tbl_ref, wd_ref, out_ref,
              buf, w_v, sem, wsem):
    k = pl.program_id(0)
    b = pl.program_id(1)
    s = k * nb + b

    def cps(kk, bb, ib):
        out = []
        for i in range(rb):
            row = lin_ref[bb * rb + i]
            out.append(pltpu.make_async_copy(
                tbl_ref.at[pl.ds(row, 1), pl.ds(kk * kc, kc)],
                buf.at[ib, pl.ds(i, 1), :], sem))
        return out

    @pl.when(b == 0)
    def _():
        pltpu.make_async_copy(
            wd_ref.at[pl.ds(woff + k * kc, kc), :], w_v, wsem).start()

    @pl.when(s == 0)
    def _():
        for cp in cps(k, b, 0):
            cp.start()

    @pl.when(s + 1 < nk * nb)
    def _():
        sn = s + 1
        for cp in cps(sn // nb, sn % nb, sn % 2):
            cp.start()

    def drain(ib):
        pltpu.make_async_copy(
            tbl_ref.at[pl.ds(0, rb), pl.ds(0, kc)], buf.at[ib], sem).wait()

    @pl.when(s % 2 == 0)
    def _():
        drain(0)

    @pl.when(s % 2 == 1)
    def _():
        drain(1)

    @pl.when(b == 0)
    def _():
        pltpu.make_async_copy(
            wd_ref.at[pl.ds(woff + k * kc, kc), :], w_v, wsem).wait()

    w = w_v[...]

    @pl.when(s % 2 == 0)
    def _():
        out_ref[0] = jnp.dot(jnp.maximum(buf[0], 0.0), w,
                             preferred_element_type=jnp.float32)

    @pl.when(s % 2 == 1)
    def _():
        out_ref[0] = jnp.dot(jnp.maximum(buf[1], 0.0), w,
                             preferred_element_type=jnp.float32)


def _gather_matmul(lin, tbl2d, w_down, woff, nk, kc, rb):
    nb = _B // rb
    grid_spec = pltpu.PrefetchScalarGridSpec(
        num_scalar_prefetch=1,
        grid=(nk, nb),
        in_specs=[
            pl.BlockSpec(memory_space=pltpu.MemorySpace.HBM),
            pl.BlockSpec(memory_space=pltpu.MemorySpace.HBM),
        ],
        out_specs=pl.BlockSpec((1, rb, 128), lambda k, b, lin_: (k, b, 0)),
        scratch_shapes=[
            pltpu.VMEM((2, rb, kc), jnp.float32),
            pltpu.VMEM((kc, 128), jnp.float32),
            pltpu.SemaphoreType.DMA,
            pltpu.SemaphoreType.DMA,
        ],
    )
    fn = pl.pallas_call(
        functools.partial(_gmm_body, nk, nb, rb, kc, woff),
        grid_spec=grid_spec,
        out_shape=jax.ShapeDtypeStruct((nk, _B, 128), jnp.float32),
        compiler_params=pltpu.CompilerParams(
            vmem_limit_bytes=56 * 1024 * 1024),
        interpret=_INTERPRET,
    )
    return fn(lin, tbl2d, w_down)


def _tail_body(h3p_ref, h2_ref, f0T_ref, nfT_ref, tsraw_ref,
               w0_ref, wtm_ref, wt0_ref, wtp_ref, wnf_ref,
               dir_ref, t_ref, wmid_ref, lng_ref, lnb_ref, wfin_ref,
               bfin_ref, out_ref):
    f32 = jnp.float32

    def rmm(x, w_ref):
        return jnp.dot(jnp.maximum(x, 0.0), w_ref[...],
                       preferred_element_type=f32)

    def rtmm(xT_ref, w_ref):
        return lax.dot_general(jnp.maximum(xT_ref[...], 0.0), w_ref[...],
                               (((0,), (0,)), ((), ())),
                               preferred_element_type=f32)

    t64 = (t_ref[...] * 63.0).astype(jnp.int32)
    wrap = t64 == 0
    raw0 = tsraw_ref[:, 0, :]
    raw1 = tsraw_ref[:, 1, :]
    raw2 = tsraw_ref[:, 2, :]
    raw3 = tsraw_ref[:, 3, :]
    tfm = jnp.where(wrap, raw3, raw0)
    tf0 = jnp.where(wrap, raw0, raw1)
    tfp = jnp.where(wrap, raw1, raw2)

    h3 = h3p_ref[0]
    for kk in range(1, h3p_ref.shape[0]):
        h3 = h3 + h3p_ref[kk]
    h = (h3 + h2_ref[...]
         + rtmm(f0T_ref, w0_ref) + rtmm(nfT_ref, wnf_ref)
         + rmm(tfm, wtm_ref) + rmm(tf0, wt0_ref) + rmm(tfp, wtp_ref))
    wmid = wmid_ref[...]
    ff = (jnp.dot(h, wmid[0:128, :], preferred_element_type=f32)
          + jnp.dot(dir_ref[...], wmid[128:131, :], preferred_element_type=f32)
          + t_ref[...] * wmid[131, :][None, :])
    mu = jnp.mean(ff, axis=-1, keepdims=True)
    d = ff - mu
    var = jnp.mean(d * d, axis=-1, keepdims=True)
    ffn = d * lax.rsqrt(var + 1e-5) * lng_ref[...] + lnb_ref[...]
    out_ref[...] = (jnp.dot(ffn, wfin_ref[...], preferred_element_type=f32)
                    + bfin_ref[...])


def _tail(h3p, h2, f0T, nfT, tsraw, w0, wtm, wt0, wtp, wnf,
          dirs, t, wmid, lng, lnb, wfin, bfin):
    fn = pl.pallas_call(
        _tail_body,
        out_shape=jax.ShapeDtypeStruct((_B, 4), jnp.float32),
        interpret=_INTERPRET,
    )
    return fn(h3p, h2, f0T, nfT, tsraw, w0, wtm, wt0, wtp, wnf,
              dirs, t, wmid, lng, lnb, wfin, bfin)


def kernel(pos, dir, t, table0, table1, table2, table3, ts_table1, ts_table2,
           W_down, W_mid, ln_g, ln_b, W_fin, b_fin):
    px = pos[:, 0]
    py = pos[:, 1]
    pz = pos[:, 2]
    t0f = table0.transpose(0, 1, 3, 2).reshape(-1)
    ts2t = ts_table2.transpose(0, 1, 2, 4, 3).reshape(128 * 128 * 128 * 4, 128)
    t1r = table1.reshape(64 * 64 * 64, 512)
    t2r = table2.reshape(32 * 32 * 32, 4096)
    t3r = table3.reshape(16 * 16 * 16, 32768)
    ts1c = ts_table1.reshape(16 * 16 * 16, 64, 64)

    idx = _idx_call(pos.T, t.reshape(1, _B))
    lin1 = idx[0]
    lin2 = idx[1]
    lin3 = idx[2]
    srow = idx[3]

    f0g, nfT = _sc_gather(px, py, pz, t, t0f, ts2t)
    f0T = f0g.reshape(_NW, 64, _BPW).transpose(1, 0, 2).reshape(64, _B)

    h2, tsraw = _a2_call(lin2, lin1, lin3, srow, t2r, t1r, ts1c, W_down)
    h3p = _gather_matmul(lin3, t3r, W_down, woff=4672, nk=1, kc=32768, rb=64)

    return _tail(
        h3p, h2, f0T, nfT, tsraw,
        W_down[0:64],
        W_down[37440:37504], W_down[37504:37568], W_down[37568:37632],
        W_down[37632:37848],
        dir, t.reshape(_B, 1), W_mid, ln_g.reshape(1, 132),
        ln_b.reshape(1, 132), W_fin, b_fin.reshape(1, 4))


# speedup vs baseline: 17.1319x; 1.0179x over previous
import functools

import jax
import jax.numpy as jnp
from jax import lax
from jax.experimental import pallas as pl
from jax.experimental.pallas import tpu as pltpu
from jax.experimental.pallas import tpu_sc as plsc

_B = 1024
_NC, _NS, _LANES = 2, 16, 16
_NW = _NC * _NS
_BPW = _B // _NW

_INTERPRET = False

_SPATIAL_OFFS = [(dx, dy, dz) for dx in (-1, 0, 1) for dy in (-1, 0, 1)
                 for dz in (-1, 0, 1)]


def _sc_body(px_h, py_h, pz_h, t_h, t0f_h, ts2t_h,
             f0g_h, nfT_h,
             px_v, py_v, pz_v, t_v, tm_v, tp_v,
             widx_v, nbrg_v, f0g_v, nfw_v, nfT_v, sem, sem2):
    wid = lax.axis_index("s") * _NC + lax.axis_index("c")
    base = wid * _BPW

    pltpu.sync_copy(px_h.at[pl.ds(base, _BPW)], px_v)
    pltpu.sync_copy(py_h.at[pl.ds(base, _BPW)], py_v)
    pltpu.sync_copy(pz_h.at[pl.ds(base, _BPW)], pz_v)
    pltpu.sync_copy(t_h.at[pl.ds(base, _BPW)], t_v)

    for c in range(_BPW // _LANES):
        sl = pl.ds(c * _LANES, _LANES)
        px = px_v[sl]
        py = py_v[sl]
        pz = pz_v[sl]
        tt = t_v[sl]
        x0 = (px * 127.0).astype(jnp.int32)
        y0 = (py * 127.0).astype(jnp.int32)
        z0 = (pz * 127.0).astype(jnp.int32)
        x3 = (px * 15.0).astype(jnp.int32)
        y3 = (py * 15.0).astype(jnp.int32)
        z3 = (pz * 15.0).astype(jnp.int32)
        t127 = (tt * 127.0).astype(jnp.int32)
        tm_v[sl] = (t127 + 127) & 127
        tp_v[sl] = (t127 + 129) & 127

        base0 = (x0 * 128 + y0) * 8192 + z0
        for f in range(64):
            p = f * _BPW + c * _LANES
            widx_v[p >> 7, pl.ds(p & 127, _LANES)] = base0 + f * 128

        xs = {-1: (x3 + 127) & 127, 0: x3, 1: (x3 + 129) & 127}
        ys = {-1: (y3 + 127) & 127, 0: y3, 1: (y3 + 129) & 127}
        zs = {-1: (z3 + 127) & 127, 0: z3, 1: (z3 + 129) & 127}
        for s, (dx, dy, dz) in enumerate(_SPATIAL_OFFS):
            spb = ((xs[dx] * 128 + ys[dy]) * 128 + zs[dz]) * 4
            for f in range(4):
                nbrg_v[s, pl.ds(f * _BPW + c * _LANES, _LANES)] = spb + f

    iot = lax.iota(jnp.int32, _LANES)
    f0cps = [pltpu.async_copy(t0f_h.at[widx_v.at[r]],
                              f0g_v.at[pl.ds(r * 128, 128)], sem2)
             for r in range(16)]

    _D = 5

    def nf_fire(s):
        return pltpu.async_copy(ts2t_h.at[nbrg_v.at[s]], nfw_v.at[s % _D],
                                sem)

    def nf_extract(s):
        ib = s % _D
        for c in range(_BPW // _LANES):
            tm = tm_v[pl.ds(c * _LANES, _LANES)]
            tp = tp_v[pl.ds(c * _LANES, _LANES)]
            for f in range(4):
                rows = f * _BPW + c * _LANES + iot
                vm = plsc.load_gather(nfw_v.at[ib], [rows, tm])
                vp = plsc.load_gather(nfw_v.at[ib], [rows, tp])
                nfT_v[(s * 2 + 0) * 4 + f, pl.ds(c * _LANES, _LANES)] = vm
                nfT_v[(s * 2 + 1) * 4 + f, pl.ds(c * _LANES, _LANES)] = vp

    cps = {}
    for s in range(_D - 1):
        cps[s] = nf_fire(s)
    for s in range(27):
        if s + _D - 1 < 27:
            cps[s + _D - 1] = nf_fire(s + _D - 1)
        cps[s].wait()
        nf_extract(s)

    for cp in f0cps:
        cp.wait()

    pltpu.sync_copy(f0g_v, f0g_h.at[wid])
    pltpu.sync_copy(nfT_v, nfT_h.at[:, pl.ds(base, _BPW)])


def _sc_gather(px, py, pz, t, t0f, ts2t):
    f32, i32 = jnp.float32, jnp.int32
    out_type = (
        jax.ShapeDtypeStruct((_NW, 64 * _BPW), f32),
        jax.ShapeDtypeStruct((216, _B), f32),
    )
    scratch = [
        pltpu.VMEM((_BPW,), f32), pltpu.VMEM((_BPW,), f32),
        pltpu.VMEM((_BPW,), f32), pltpu.VMEM((_BPW,), f32),
        pltpu.VMEM((_BPW,), i32), pltpu.VMEM((_BPW,), i32),
        pltpu.VMEM((16, 128), i32),
        pltpu.VMEM((27, 128), i32),
        pltpu.VMEM((64 * _BPW,), f32),
        pltpu.VMEM((5, 128, 128), f32),
        pltpu.VMEM((216, _BPW), f32),
        pltpu.SemaphoreType.DMA, pltpu.SemaphoreType.DMA,
    ]
    mesh = plsc.VectorSubcoreMesh(core_axis_name="c", subcore_axis_name="s",
                                  num_cores=_NC, num_subcores=_NS)
    fn = pl.kernel(_sc_body, out_type, mesh=mesh, scratch_types=scratch,
                   compiler_params=pltpu.CompilerParams(
                       use_tc_tiling_on_sc=False, needs_layout_passes=False),
                   interpret=_INTERPRET)
    return fn(px, py, pz, t, t0f, ts2t)


def _idx_body(posT_ref, t_ref, out_ref):
    i32 = jnp.int32
    px = posT_ref[0:1, :]
    py = posT_ref[1:2, :]
    pz = posT_ref[2:3, :]
    x1 = (px * 63.0).astype(i32)
    y1 = (py * 63.0).astype(i32)
    z1 = (pz * 63.0).astype(i32)
    x2 = (px * 31.0).astype(i32)
    y2 = (py * 31.0).astype(i32)
    z2 = (pz * 31.0).astype(i32)
    x3 = (px * 15.0).astype(i32)
    y3 = (py * 15.0).astype(i32)
    z3 = (pz * 15.0).astype(i32)
    t64 = (t_ref[...] * 63.0).astype(i32)
    out_ref[0:1, :] = (x1 * 64 + y1) * 64 + z1
    out_ref[1:2, :] = (x2 * 32 + y2) * 32 + z2
    out_ref[2:3, :] = (x3 * 16 + y3) * 16 + z3
    out_ref[3:4, :] = jnp.minimum(jnp.maximum(t64 - 1, 0), 61)


def _idx_call(posT, t1):
    fn = pl.pallas_call(
        _idx_body,
        out_shape=jax.ShapeDtypeStruct((4, _B), jnp.int32),
        interpret=_INTERPRET,
    )
    return fn(posT, t1)


def _a2_body(lin2_ref, lin1_ref, lin3_ref, srow_ref,
             t2_ref, t1_ref, ts1_ref, wd_ref,
             h2_ref, tsraw_ref, f2b, f1b, tsb, w2_v, w1_v, sem, wsem):
    rb = f2b.shape[1]
    b = pl.program_id(0)
    nb = pl.num_programs(0)

    def wcps():
        return [
            pltpu.make_async_copy(wd_ref.at[pl.ds(576, 4096), :], w2_v, wsem),
            pltpu.make_async_copy(wd_ref.at[pl.ds(64, 512), :], w1_v, wsem),
        ]

    @pl.when(b == 0)
    def _():
        for cp in wcps():
            cp.start()

    def cps(bb, ib):
        out = []
        for i in range(rb):
            r2 = lin2_ref[bb * rb + i]
            r1 = lin1_ref[bb * rb + i]
            c3 = lin3_ref[bb * rb + i]
            sr = srow_ref[bb * rb + i]
            out.append(pltpu.make_async_copy(
                t2_ref.at[pl.ds(r2, 1), :], f2b.at[ib, pl.ds(i, 1), :], sem))
            out.append(pltpu.make_async_copy(
                t1_ref.at[pl.ds(r1, 1), :], f1b.at[ib, pl.ds(i, 1), :], sem))
            out.append(pltpu.make_async_copy(
                ts1_ref.at[pl.ds(c3, 1), pl.ds(sr, 3), :],
                tsb.at[ib, pl.ds(i, 1), pl.ds(0, 3), :], sem))
            out.append(pltpu.make_async_copy(
                ts1_ref.at[pl.ds(c3, 1), pl.ds(63, 1), :],
                tsb.at[ib, pl.ds(i, 1), pl.ds(3, 1), :], sem))
        return out

    @pl.when(b == 0)
    def _():
        for cp in cps(b, 0):
            cp.start()

    @pl.when(b + 1 < nb)
    def _():
        for cp in cps(b + 1, (b + 1) % 2):
            cp.start()

    def drain(ib):
        pltpu.make_async_copy(
            t2_ref.at[pl.ds(0, rb), :], f2b.at[ib], sem).wait()
        pltpu.make_async_copy(
            t1_ref.at[pl.ds(0, rb), :], f1b.at[ib], sem).wait()
        pltpu.make_async_copy(
            ts1_ref.at[pl.ds(0, rb), pl.ds(0, 4), :], tsb.at[ib], sem).wait()

    @pl.when(b % 2 == 0)
    def _():
        drain(0)

    @pl.when(b % 2 == 1)
    def _():
        drain(1)

    @pl.when(b == 0)
    def _():
        for cp in wcps():
            cp.wait()

    def compute(ib):
        h2_ref[...] = (
            jnp.dot(jnp.maximum(f2b[ib], 0.0), w2_v[...],
                    preferred_element_type=jnp.float32)
            + jnp.dot(jnp.maximum(f1b[ib], 0.0), w1_v[...],
                      preferred_element_type=jnp.float32))
        tsraw_ref[...] = tsb[ib]

    @pl.when(b % 2 == 0)
    def _():
        compute(0)

    @pl.when(b % 2 == 1)
    def _():
        compute(1)


def _a2_call(lin2, lin1, lin3, srow, t2r, t1r, ts1c, w_down, rb=64):
    nb = _B // rb
    grid_spec = pltpu.PrefetchScalarGridSpec(
        num_scalar_prefetch=4,
        grid=(nb,),
        in_specs=[
            pl.BlockSpec(memory_space=pltpu.MemorySpace.HBM),
            pl.BlockSpec(memory_space=pltpu.MemorySpace.HBM),
            pl.BlockSpec(memory_space=pltpu.MemorySpace.HBM),
            pl.BlockSpec(memory_space=pltpu.MemorySpace.HBM),
        ],
        out_specs=[
            pl.BlockSpec((rb, 128), lambda b, *_: (b, 0)),
            pl.BlockSpec((rb, 4, 64), lambda b, *_: (b, 0, 0)),
        ],
        scratch_shapes=[
            pltpu.VMEM((2, rb, 4096), jnp.float32),
            pltpu.VMEM((2, rb, 512), jnp.float32),
            pltpu.VMEM((2, rb, 4, 64), jnp.float32),
            pltpu.VMEM((4096, 128), jnp.float32),
            pltpu.VMEM((512, 128), jnp.float32),
            pltpu.SemaphoreType.DMA,
            pltpu.SemaphoreType.DMA,
        ],
    )
    fn = pl.pallas_call(
        _a2_body,
        grid_spec=grid_spec,
        out_shape=[
            jax.ShapeDtypeStruct((_B, 128), jnp.float32),
            jax.ShapeDtypeStruct((_B, 4, 64), jnp.float32),
        ],
        interpret=_INTERPRET,
    )
    return fn(lin2, lin1, lin3, srow, t2r, t1r, ts1c, w_down)


def _gmm_body(nk, nb, rb, kc, woff, lin_ref, tbl_ref, wd_ref, out_ref,
              buf, w_v, sem, wsem):
    k = pl.program_id(0)
    b = pl.program_id(1)
    s = k * nb + b

    def cps(kk, bb, ib):
        out = []
        for i in range(rb):
            row = lin_ref[bb * rb + i]
            out.append(pltpu.make_async_copy(
                tbl_ref.at[pl.ds(row, 1), pl.ds(kk * kc, kc)],
                buf.at[ib, pl.ds(i, 1), :], sem))
        return out

    @pl.when(b == 0)
    def _():
        pltpu.make_async_copy(
            wd_ref.at[pl.ds(woff + k * kc, kc), :], w_v, wsem).start()

    @pl.when(s == 0)
    def _():
        for cp in cps(k, b, 0):
            cp.start()

    @pl.when(s + 1 < nk * nb)
    def _():
        sn = s + 1
        for cp in cps(sn // nb, sn % nb, sn % 2):
            cp.start()

    def drain(ib):
        pltpu.make_async_copy(
            tbl_ref.at[pl.ds(0, rb), pl.ds(0, kc)], buf.at[ib], sem).wait()

    @pl.when(s % 2 == 0)
    def _():
        drain(0)

    @pl.when(s % 2 == 1)
    def _():
        drain(1)

    @pl.when(b == 0)
    def _():
        pltpu.make_async_copy(
            wd_ref.at[pl.ds(woff + k * kc, kc), :], w_v, wsem).wait()

    w = w_v[...]

    @pl.when(s % 2 == 0)
    def _():
        out_ref[0] = jnp.dot(jnp.maximum(buf[0], 0.0), w,
                             preferred_element_type=jnp.float32)

    @pl.when(s % 2 == 1)
    def _():
        out_ref[0] = jnp.dot(jnp.maximum(buf[1], 0.0), w,
                             preferred_element_type=jnp.float32)


def _gather_matmul(lin, tbl2d, w_down, woff, nk, kc, rb):
    nb = _B // rb
    grid_spec = pltpu.PrefetchScalarGridSpec(
        num_scalar_prefetch=1,
        grid=(nk, nb),
        in_specs=[
            pl.BlockSpec(memory_space=pltpu.MemorySpace.HBM),
            pl.BlockSpec(memory_space=pltpu.MemorySpace.HBM),
        ],
        out_specs=pl.BlockSpec((1, rb, 128), lambda k, b, lin_: (k, b, 0)),
        scratch_shapes=[
            pltpu.VMEM((2, rb, kc), jnp.float32),
            pltpu.VMEM((kc, 128), jnp.float32),
            pltpu.SemaphoreType.DMA,
            pltpu.SemaphoreType.DMA,
        ],
    )
    fn = pl.pallas_call(
        functools.partial(_gmm_body, nk, nb, rb, kc, woff),
        grid_spec=grid_spec,
        out_shape=jax.ShapeDtypeStruct((nk, _B, 128), jnp.float32),
        compiler_params=pltpu.CompilerParams(
            vmem_limit_bytes=56 * 1024 * 1024),
        interpret=_INTERPRET,
    )
    return fn(lin, tbl2d, w_down)


def _tail_body(h3p_ref, h2_ref, f0T_ref, nfT_ref, tsraw_ref,
               w0_ref, wtm_ref, wt0_ref, wtp_ref, wnf_ref,
               dir_ref, t_ref, wmid_ref, lng_ref, lnb_ref, wfin_ref,
               bfin_ref, out_ref):
    f32 = jnp.float32

    def rmm(x, w_ref):
        return jnp.dot(jnp.maximum(x, 0.0), w_ref[...],
                       preferred_element_type=f32)

    def rtmm(xT_ref, w_ref):
        return lax.dot_general(jnp.maximum(xT_ref[...], 0.0), w_ref[...],
                               (((0,), (0,)), ((), ())),
                               preferred_element_type=f32)

    t64 = (t_ref[...] * 63.0).astype(jnp.int32)
    wrap = t64 == 0
    raw0 = tsraw_ref[:, 0, :]
    raw1 = tsraw_ref[:, 1, :]
    raw2 = tsraw_ref[:, 2, :]
    raw3 = tsraw_ref[:, 3, :]
    tfm = jnp.where(wrap, raw3, raw0)
    tf0 = jnp.where(wrap, raw0, raw1)
    tfp = jnp.where(wrap, raw1, raw2)

    h3 = h3p_ref[0]
    for kk in range(1, h3p_ref.shape[0]):
        h3 = h3 + h3p_ref[kk]
    h = (h3 + h2_ref[...]
         + rtmm(f0T_ref, w0_ref) + rtmm(nfT_ref, wnf_ref)
         + rmm(tfm, wtm_ref) + rmm(tf0, wt0_ref) + rmm(tfp, wtp_ref))
    wmid = wmid_ref[...]
    ff = (jnp.dot(h, wmid[0:128, :], preferred_element_type=f32)
          + jnp.dot(dir_ref[...], wmid[128:131, :], preferred_element_type=f32)
          + t_ref[...] * wmid[131, :][None, :])
    mu = jnp.mean(ff, axis=-1, keepdims=True)
    d = ff - mu
    var = jnp.mean(d * d, axis=-1, keepdims=True)
    ffn = d * lax.rsqrt(var + 1e-5) * lng_ref[...] + lnb_ref[...]
    out_ref[...] = (jnp.dot(ffn, wfin_ref[...], preferred_element_type=f32)
                    + bfin_ref[...])


def _tail(h3p, h2, f0T, nfT, tsraw, w0, wtm, wt0, wtp, wnf,
          dirs, t, wmid, lng, lnb, wfin, bfin):
    fn = pl.pallas_call(
        _tail_body,
        out_shape=jax.ShapeDtypeStruct((_B, 4), jnp.float32),
        interpret=_INTERPRET,
    )
    return fn(h3p, h2, f0T, nfT, tsraw, w0, wtm, wt0, wtp, wnf,
              dirs, t, wmid, lng, lnb, wfin, bfin)


def kernel(pos, dir, t, table0, table1, table2, table3, ts_table1, ts_table2,
           W_down, W_mid, ln_g, ln_b, W_fin, b_fin):
    px = pos[:, 0]
    py = pos[:, 1]
    pz = pos[:, 2]
    t0f = table0.transpose(0, 1, 3, 2).reshape(-1)
    ts2t = ts_table2.transpose(0, 1, 2, 4, 3).reshape(128 * 128 * 128 * 4, 128)
    t1r = table1.reshape(64 * 64 * 64, 512)
    t2r = table2.reshape(32 * 32 * 32, 4096)
    t3r = table3.reshape(16 * 16 * 16, 32768)
    ts1c = ts_table1.reshape(16 * 16 * 16, 64, 64)

    idx = _idx_call(pos.T, t.reshape(1, _B))
    lin1 = idx[0]
    lin2 = idx[1]
    lin3 = idx[2]
    srow = idx[3]

    f0g, nfT = _sc_gather(px, py, pz, t, t0f, ts2t)
    f0T = f0g.reshape(_NW, 64, _BPW).transpose(1, 0, 2).reshape(64, _B)

    h2, tsraw = _a2_call(lin2, lin1, lin3, srow, t2r, t1r, ts1c, W_down,
                         rb=128)
    h3p = _gather_matmul(lin3, t3r, W_down, woff=4672, nk=1, kc=32768, rb=64)

    return _tail(
        h3p, h2, f0T, nfT, tsraw,
        W_down[0:64],
        W_down[37440:37504], W_down[37504:37568], W_down[37568:37632],
        W_down[37632:37848],
        dir, t.reshape(_B, 1), W_mid, ln_g.reshape(1, 132),
        ln_b.reshape(1, 132), W_fin, b_fin.reshape(1, 4))
